# baseline (speedup 1.0000x reference)
import functools

import jax
import jax.numpy as jnp
from jax import lax
from jax.experimental import pallas as pl
from jax.experimental.pallas import tpu as pltpu
from jax.experimental.pallas import tpu_sc as plsc

D = 128
H = 8
DH = 16
HD = 64
WA = 128
NC = 2
NS = 16
NW = NC * NS
CH = 80
RB = 2000


def _sc_gather(KV, Q, src, dst):
    N, DKV = KV.shape
    NE = src.shape[0]
    per_w = NE // NW
    iters = per_w // CH
    pairs = (iters - 1) // 2
    mesh = plsc.VectorSubcoreMesh(core_axis_name="c", subcore_axis_name="s")

    @functools.partial(
        pl.kernel,
        out_type=[jax.ShapeDtypeStruct((NE, DKV), jnp.float32),
                  jax.ShapeDtypeStruct((NE, D), jnp.float32)],
        mesh=mesh,
        scratch_types=[
            pltpu.VMEM((per_w,), jnp.int32),
            pltpu.VMEM((per_w,), jnp.int32),
            pltpu.VMEM((CH, DKV), jnp.float32),
            pltpu.VMEM((CH, DKV), jnp.float32),
            pltpu.VMEM((CH, D), jnp.float32),
            pltpu.VMEM((CH, D), jnp.float32),
            pltpu.SemaphoreType.DMA,
            pltpu.SemaphoreType.DMA,
            pltpu.SemaphoreType.DMA,
            pltpu.SemaphoreType.DMA,
        ],
    )
    def gather_k(kv_hbm, q_hbm, src_hbm, dst_hbm, kv_out, q_out,
                 idx_s, idx_d, bkva, bkvb, bqa, bqb,
                 ska, skb, sqa, sqb):
        wid = lax.axis_index("s") * NC + lax.axis_index("c")
        base = wid * per_w
        pltpu.sync_copy(src_hbm.at[pl.ds(base, per_w)], idx_s)
        pltpu.sync_copy(dst_hbm.at[pl.ds(base, per_w)], idx_d)

        def fire(j, bkv, bq, skv, sq):
            pltpu.async_copy(kv_hbm.at[idx_s.at[pl.ds(j * CH, CH)]], bkv, skv)
            pltpu.async_copy(q_hbm.at[idx_d.at[pl.ds(j * CH, CH)]], bq, sq)

        def finish(j, bkv, bq, skv, sq):
            off = base + j * CH
            pltpu.make_async_copy(
                kv_hbm.at[idx_s.at[pl.ds(j * CH, CH)]], bkv, skv).wait()
            pltpu.make_async_copy(
                q_hbm.at[idx_d.at[pl.ds(j * CH, CH)]], bq, sq).wait()
            pltpu.sync_copy(bkv, kv_out.at[pl.ds(off, CH)])
            pltpu.sync_copy(bq, q_out.at[pl.ds(off, CH)])

        fire(0, bkva, bqa, ska, sqa)

        def body(i, carry):
            j = 2 * i
            fire(j + 1, bkvb, bqb, skb, sqb)
            finish(j, bkva, bqa, ska, sqa)
            fire(j + 2, bkva, bqa, ska, sqa)
            finish(j + 1, bkvb, bqb, skb, sqb)
            return carry

        lax.fori_loop(0, pairs, body, 0)
        finish(iters - 1, bkva, bqa, ska, sqa)

    return gather_k(KV, Q, src, dst)


def _sc_scatter(data, dst, N):
    NE, W = data.shape
    per_w = NE // NW
    iters = per_w // CH
    RC = 80
    RT = 640
    zeros_w = jnp.zeros((RC, W), jnp.float32)
    mesh = plsc.VectorSubcoreMesh(core_axis_name="c", subcore_axis_name="s")

    @functools.partial(
        pl.kernel,
        out_type=jax.ShapeDtypeStruct((NC, N, W), jnp.float32),
        mesh=mesh,
        scratch_types=[
            pltpu.VMEM((per_w,), jnp.int32),
            pltpu.VMEM((CH, W), jnp.float32),
            pltpu.VMEM((CH, W), jnp.float32),
            pltpu.VMEM((RC, W), jnp.float32),
            pltpu.VMEM_SHARED((N, W), jnp.float32),
            pltpu.SemaphoreType.DMA,
            pltpu.SemaphoreType.DMA,
            pltpu.SemaphoreType.DMA,
            pltpu.SemaphoreType.DMA,
        ],
    )
    def scatter_k(d_hbm, dst_hbm, z_hbm, out, idx, eba, ebb, ib,
                  acc, sfa, sfb, saa, sab):
        cid = lax.axis_index("c")
        sid = lax.axis_index("s")
        wid = sid * NC + cid
        row0 = sid * RT
        n_chunks = jnp.minimum(N - row0, RT) // RC
        base = wid * per_w

        pltpu.sync_copy(z_hbm, ib)

        def initj(j, carry):
            pltpu.sync_copy(ib, acc.at[pl.ds(row0 + j * RC, RC)])
            return carry

        lax.fori_loop(0, n_chunks, initj, 0)
        pltpu.sync_copy(dst_hbm.at[pl.ds(base, per_w)], idx)
        plsc.subcore_barrier()

        def idx_sl(j):
            return idx.at[pl.ds(j * CH, CH)]

        def fetch(j, eb, sf):
            pltpu.async_copy(d_hbm.at[pl.ds(base + j * CH, CH)], eb, sf)

        def wait_fetch(j, eb, sf):
            pltpu.make_async_copy(
                d_hbm.at[pl.ds(base + j * CH, CH)], eb, sf).wait()

        def add(j, eb, sa):
            pltpu.async_copy(eb, acc.at[idx_sl(j)], sa, add=True)

        def wait_add(j, eb, sa):
            pltpu.make_async_copy(eb, acc.at[idx_sl(j)], sa).wait()

        fetch(0, eba, sfa)
        fetch(1, ebb, sfb)

        def body(i, carry):
            j = 2 * i
            wait_fetch(j, eba, sfa)
            add(j, eba, saa)
            wait_fetch(j + 1, ebb, sfb)
            add(j + 1, ebb, sab)
            wait_add(j, eba, saa)
            fetch(j + 2, eba, sfa)
            wait_add(j + 1, ebb, sab)
            fetch(j + 3, ebb, sfb)
            return carry

        lax.fori_loop(0, (iters - 3) // 2, body, 0)
        j = iters - 3
        wait_fetch(j, eba, sfa)
        add(j, eba, saa)
        wait_add(j, eba, saa)
        fetch(j + 2, eba, sfa)
        wait_fetch(j + 1, ebb, sfb)
        add(j + 1, ebb, sab)
        wait_fetch(j + 2, eba, sfa)
        add(j + 2, eba, saa)
        wait_add(j + 1, ebb, sab)
        wait_add(j + 2, eba, saa)
        plsc.subcore_barrier()

        def outj(j, carry):
            r = row0 + j * RC
            pltpu.sync_copy(acc.at[pl.ds(r, RC)], ib)
            pltpu.sync_copy(ib, out.at[cid, pl.ds(r, RC)])
            return carry

        lax.fori_loop(0, n_chunks, outj, 0)

    return scatter_k(data, dst, zeros_w)


def _dot(a, b):
    return jnp.dot(a, b, preferred_element_type=jnp.float32)


def _full(shape=None):
    return pl.BlockSpec(memory_space=pltpu.ANY) if shape is None else \
        pl.BlockSpec(shape, lambda i: (0,) * len(shape))


def _rows(shape):
    return pl.BlockSpec(shape, lambda i: (i,) + (0,) * (len(shape) - 1))


def _tc_embed_qkv(h, embW, embb, qW, qb, kW, kb, vW, vb):
    N = h.shape[0]

    def body(h_ref, ew_ref, eb_ref, qw_ref, qb_ref, kw_ref, kb_ref,
             vw_ref, vb_ref, hh_ref, q_ref, kv_ref):
        hh = _dot(h_ref[...], ew_ref[...]) + eb_ref[...]
        hh_ref[...] = hh
        q_ref[...] = _dot(hh, qw_ref[...]) + qb_ref[...]
        kv_ref[...] = jnp.concatenate(
            [_dot(hh, kw_ref[...]) + kb_ref[...],
             _dot(hh, vw_ref[...]) + vb_ref[...]], axis=1)

    return pl.pallas_call(
        body,
        out_shape=[jax.ShapeDtypeStruct((N, D), jnp.float32)] * 2
        + [jax.ShapeDtypeStruct((N, 2 * D), jnp.float32)],
    )(h, embW, embb, qW, qb, kW, kb, vW, vb)


def _tc_edge_full(kvsrc, qdst, e, embW, embb, eW, eb, oeW, oeb,
                  ones_hb, ones_hb_t):
    NE = kvsrc.shape[0]
    F = e.shape[1]
    grid = NE // RB

    def body(kv_ref, qd_ref, e_ref, ew_ref, ebias_ref, pw_ref, pb_ref,
             ow_ref, ob_ref, hb_ref, hbt_ref,
             wva_ref, wvb_ref, epre_ref, st_ref):
        ee = _dot(e_ref[...], ew_ref[...]) + ebias_ref[...]
        ep = _dot(ee, pw_ref[...]) + pb_ref[...]
        score = kv_ref[:, :D] * qd_ref[...] * ep * 0.25
        ssum = _dot(score, hbt_ref[...])
        attn = jnp.exp(jnp.clip(ssum, -5.0, 5.0))
        attnb = _dot(attn, hb_ref[...])
        wv = kv_ref[:, D:] * attnb
        pad = jnp.zeros((RB, WA - HD - H), jnp.float32)
        wva_ref[...] = jnp.concatenate([wv[:, :HD], attn, pad], axis=1)
        wvb_ref[...] = jnp.concatenate([wv[:, HD:], attn, pad], axis=1)
        epre = ee + _dot(score, ow_ref[...]) + ob_ref[...]
        epre_ref[...] = epre

        @pl.when(pl.program_id(0) == 0)
        def _():
            st_ref[...] = jnp.zeros_like(st_ref)

        s = jnp.sum(epre, axis=0)
        ss = jnp.sum(epre * epre, axis=0)
        st_ref[...] += jnp.concatenate(
            [s[None], ss[None], jnp.zeros((6, D), jnp.float32)], axis=0)

    return pl.pallas_call(
        body,
        grid=(grid,),
        in_specs=[_rows((RB, 2 * D)), _rows((RB, D)), _rows((RB, F)),
                  _full((F, D)), _full((1, D)), _full((D, D)), _full((1, D)),
                  _full((D, D)), _full((1, D)), _full((H, D)), _full((D, H))],
        out_specs=[_rows((RB, WA)), _rows((RB, WA)), _rows((RB, D)),
                   _full((8, D))],
        out_shape=[jax.ShapeDtypeStruct((NE, WA), jnp.float32),
                   jax.ShapeDtypeStruct((NE, WA), jnp.float32),
                   jax.ShapeDtypeStruct((NE, D), jnp.float32),
                   jax.ShapeDtypeStruct((8, D), jnp.float32)],
    )(kvsrc, qdst, e, embW, embb, eW, eb, oeW, oeb, ones_hb, ones_hb_t)


def _tc_edge_lite(kvsrc, qdst, y, stats, g2, b2, eW, eb, n_rows,
                  ones_hb, ones_hb_t):
    NE = kvsrc.shape[0]
    grid = NE // RB
    inv_n = 1.0 / float(n_rows)

    def body(kv_ref, qd_ref, y_ref, st_ref, g_ref, b_ref, pw_ref, pb_ref,
             hb_ref, hbt_ref, wva_ref, wvb_ref):
        mean = st_ref[0:1, :] * inv_n
        var = st_ref[1:2, :] * inv_n - mean * mean
        inv = lax.rsqrt(var + 1e-5)
        x = (y_ref[...] - mean) * inv * g_ref[...] + b_ref[...]
        ep = _dot(x, pw_ref[...]) + pb_ref[...]
        score = kv_ref[:, :D] * qd_ref[...] * ep * 0.25
        attn = jnp.exp(jnp.clip(_dot(score, hbt_ref[...]), -5.0, 5.0))
        attnb = _dot(attn, hb_ref[...])
        wv = kv_ref[:, D:] * attnb
        pad = jnp.zeros((RB, WA - HD - H), jnp.float32)
        wva_ref[...] = jnp.concatenate([wv[:, :HD], attn, pad], axis=1)
        wvb_ref[...] = jnp.concatenate([wv[:, HD:], attn, pad], axis=1)

    return pl.pallas_call(
        body,
        grid=(grid,),
        in_specs=[_rows((RB, 2 * D))] + [_rows((RB, D))] * 2
        + [_full((8, D)), _full((1, D)), _full((1, D)),
           _full((D, D)), _full((1, D)), _full((H, D)), _full((D, H))],
        out_specs=[_rows((RB, WA)), _rows((RB, WA))],
        out_shape=[jax.ShapeDtypeStruct((NE, WA), jnp.float32),
                   jax.ShapeDtypeStruct((NE, WA), jnp.float32)],
    )(kvsrc, qdst, y, stats, g2, b2, eW, eb, ones_hb, ones_hb_t)


def _tc_edge_ffn(epre, stats, g1, b1, w1, bb1, w2, bb2, n_rows):
    NE = epre.shape[0]
    grid = NE // RB
    inv_n = 1.0 / float(n_rows)

    def body(ep_ref, st_ref, g_ref, b_ref, w1_ref, b1_ref, w2_ref, b2_ref,
             y_ref, sy_ref):
        mean = st_ref[0:1, :] * inv_n
        var = st_ref[1:2, :] * inv_n - mean * mean
        inv = lax.rsqrt(var + 1e-5)
        x = (ep_ref[...] - mean) * inv * g_ref[...] + b_ref[...]
        hmid = jnp.maximum(_dot(x, w1_ref[...]) + b1_ref[...], 0.0)
        y = x + _dot(hmid, w2_ref[...]) + b2_ref[...]
        y_ref[...] = y

        @pl.when(pl.program_id(0) == 0)
        def _():
            sy_ref[...] = jnp.zeros_like(sy_ref)

        s = jnp.sum(y, axis=0)
        ss = jnp.sum(y * y, axis=0)
        sy_ref[...] += jnp.concatenate(
            [s[None], ss[None], jnp.zeros((6, D), jnp.float32)], axis=0)

    return pl.pallas_call(
        body,
        grid=(grid,),
        in_specs=[_rows((RB, D)), _full((8, D)), _full((1, D)), _full((1, D)),
                  _full((D, 2 * D)), _full((1, 2 * D)),
                  _full((2 * D, D)), _full((1, D))],
        out_specs=[_rows((RB, D)), _full((8, D))],
        out_shape=[jax.ShapeDtypeStruct((NE, D), jnp.float32),
                   jax.ShapeDtypeStruct((8, D), jnp.float32)],
    )(epre, stats, g1, b1, w1, bb1, w2, bb2)


def _node_update(hpa, hpb, hh_in, lp, ones_hb):
    a = hpa[0] + hpa[1]
    b = hpb[0] + hpb[1]
    wv = jnp.concatenate([a[:, :HD], b[:, :HD]], axis=1)
    z = a[:, HD:HD + H]
    r = 1.0 / (z + 1e-6)
    h_att = wv * _dot(r, ones_hb)
    h_new = _dot(h_att, lp['oh_w']) + lp['oh_b'] + hh_in
    m = jnp.mean(h_new, axis=0, keepdims=True)
    v = jnp.mean(h_new * h_new, axis=0, keepdims=True) - m * m
    h_new = (h_new - m) * lax.rsqrt(v + 1e-5) * lp['g1'] + lp['b1']
    h2 = _dot(jnp.maximum(_dot(h_new, lp['f1w']) + lp['f1b'], 0.0),
              lp['f2w']) + lp['f2b']
    h_new = h_new + h2
    m = jnp.mean(h_new, axis=0, keepdims=True)
    v = jnp.mean(h_new * h_new, axis=0, keepdims=True) - m * m
    return (h_new - m) * lax.rsqrt(v + 1e-5) * lp['g2'] + lp['b2']


def _tc_node_mid(hpart, zpart, hh_in, lw, qW, qb, kW, kb, vW, vb, ones_hb):
    N = hh_in.shape[0]

    def body(hp_ref, zp_ref, hh_ref, ow_ref, ob_ref, g1_ref, b1_ref,
             f1w_ref, f1b_ref, f2w_ref, f2b_ref, g2_ref, b2_ref,
             qw_ref, qb_ref, kw_ref, kb_ref, vw_ref, vb_ref, hb_ref,
             hh_out, q_ref, kv_ref):
        lp = dict(oh_w=ow_ref[...], oh_b=ob_ref[...], g1=g1_ref[...],
                  b1=b1_ref[...], f1w=f1w_ref[...], f1b=f1b_ref[...],
                  f2w=f2w_ref[...], f2b=f2b_ref[...], g2=g2_ref[...],
                  b2=b2_ref[...])
        hh1 = _node_update(hp_ref[...], zp_ref[...], hh_ref[...], lp,
                           hb_ref[...])
        hh_out[...] = hh1
        q_ref[...] = _dot(hh1, qw_ref[...]) + qb_ref[...]
        kv_ref[...] = jnp.concatenate(
            [_dot(hh1, kw_ref[...]) + kb_ref[...],
             _dot(hh1, vw_ref[...]) + vb_ref[...]], axis=1)

    return pl.pallas_call(
        body,
        out_shape=[jax.ShapeDtypeStruct((N, D), jnp.float32)] * 2
        + [jax.ShapeDtypeStruct((N, 2 * D), jnp.float32)],
    )(hpart, zpart, hh_in, lw['oh_w'], lw['oh_b'], lw['g1'], lw['b1'],
      lw['f1w'], lw['f1b'], lw['f2w'], lw['f2b'], lw['g2'], lw['b2'],
      qW, qb, kW, kb, vW, vb, ones_hb)


def _tc_node_head(hpart, zpart, hh_in, lw, w_top, w_bot, mlp1b, mlp2w,
                  mlp2b, vid, ones_hb, num_nodes):
    N = hh_in.shape[0]
    bs = vid.shape[0]

    def body(hp_ref, zp_ref, hh_ref, ow_ref, ob_ref, g1_ref, b1_ref,
             f1w_ref, f1b_ref, f2w_ref, f2b_ref, g2_ref, b2_ref,
             wt_ref, wb_ref, m1b_ref, m2w_ref, m2b_ref, hb_ref, vid_ref,
             out_ref, hh2_ref):
        lp = dict(oh_w=ow_ref[...], oh_b=ob_ref[...], g1=g1_ref[...],
                  b1=b1_ref[...], f1w=f1w_ref[...], f1b=f1b_ref[...],
                  f2w=f2w_ref[...], f2b=f2b_ref[...], g2=g2_ref[...],
                  b2=b2_ref[...])
        hh2 = _node_update(hp_ref[...], zp_ref[...], hh_ref[...], lp,
                           hb_ref[...])
        hh2_ref[...] = hh2
        for b in range(bs):
            row = hh2_ref[pl.ds(vid_ref[b], 1), :]
            cb = _dot(row, wt_ref[...]) + m1b_ref[...]
            blk = hh2_ref[pl.ds(b * num_nodes, num_nodes), :]
            t = jnp.maximum(_dot(blk, wb_ref[...]) + cb, 0.0)
            out_ref[pl.ds(b * num_nodes, num_nodes), :] = (
                _dot(t, m2w_ref[...]) + m2b_ref[...])

    return pl.pallas_call(
        body,
        in_specs=[pl.BlockSpec(memory_space=pltpu.VMEM)] * 19
        + [pl.BlockSpec(memory_space=pltpu.SMEM)],
        out_specs=pl.BlockSpec(memory_space=pltpu.VMEM),
        out_shape=jax.ShapeDtypeStruct((N, 1), jnp.float32),
        scratch_shapes=[pltpu.VMEM((N, D), jnp.float32)],
    )(hpart, zpart, hh_in, lw['oh_w'], lw['oh_b'], lw['g1'], lw['b1'],
      lw['f1w'], lw['f1b'], lw['f2w'], lw['f2b'], lw['g2'], lw['b2'],
      w_top, w_bot, mlp1b, mlp2w, mlp2b, ones_hb, vid)


def _row(x):
    return x.reshape(1, -1)


def _layer_w(lp):
    return dict(oh_w=lp['Oh_W'], oh_b=_row(lp['Oh_b']),
                g1=_row(lp['bn1h_g']), b1=_row(lp['bn1h_b']),
                f1w=lp['ffh1_W'], f1b=_row(lp['ffh1_b']),
                f2w=lp['ffh2_W'], f2b=_row(lp['ffh2_b']),
                g2=_row(lp['bn2h_g']), b2=_row(lp['bn2h_b']))


def kernel(h, e, params, edge_index, vehicle_node_id, batch_size):
    src = edge_index[0]
    dst = edge_index[1]
    N = h.shape[0]
    NE = src.shape[0]
    bs = vehicle_node_id.shape[0]
    num_nodes = N // bs
    L1, L2 = params['layers']

    ones_hb = jnp.repeat(jnp.eye(H, dtype=jnp.float32), DH, axis=1)
    ones_hb_t = ones_hb.T

    hh0, q1, kv1 = _tc_embed_qkv(
        h, params['emb_h_W'], _row(params['emb_h_b']),
        L1['Q_W'], _row(L1['Q_b']), L1['K_W'], _row(L1['K_b']),
        L1['V_W'], _row(L1['V_b']))

    kvsrc1, qdst1 = _sc_gather(kv1, q1, src, dst)
    wv1a, wv1b, epre1, st1 = _tc_edge_full(
        kvsrc1, qdst1, e, params['emb_e_W'], _row(params['emb_e_b']),
        L1['E_W'], _row(L1['E_b']), L1['Oe_W'], _row(L1['Oe_b']),
        ones_hb, ones_hb_t)
    hpa1 = _sc_scatter(wv1a, dst, N)
    hpb1 = _sc_scatter(wv1b, dst, N)
    hh1, q2, kv2 = _tc_node_mid(
        hpa1, hpb1, hh0, _layer_w(L1),
        L2['Q_W'], _row(L2['Q_b']), L2['K_W'], _row(L2['K_b']),
        L2['V_W'], _row(L2['V_b']), ones_hb)
    y1, sty1 = _tc_edge_ffn(
        epre1, st1, _row(L1['bn1e_g']), _row(L1['bn1e_b']),
        L1['ffe1_W'], _row(L1['ffe1_b']), L1['ffe2_W'], _row(L1['ffe2_b']),
        NE)

    kvsrc2, qdst2 = _sc_gather(kv2, q2, src, dst)
    wv2a, wv2b = _tc_edge_lite(
        kvsrc2, qdst2, y1, sty1, _row(L1['bn2e_g']), _row(L1['bn2e_b']),
        L2['E_W'], _row(L2['E_b']), NE, ones_hb, ones_hb_t)
    hpa2 = _sc_scatter(wv2a, dst, N)
    hpb2 = _sc_scatter(wv2b, dst, N)

    vid = (vehicle_node_id.astype(jnp.int32)
           + jnp.arange(bs, dtype=jnp.int32) * num_nodes
           + (jnp.asarray(batch_size, jnp.int32) - jnp.int32(bs)))
    policy = _tc_node_head(
        hpa2, hpb2, hh1, _layer_w(L2),
        params['mlp1_W'][:D], params['mlp1_W'][D:], _row(params['mlp1_b']),
        params['mlp2_W'], _row(params['mlp2_b']), vid, ones_hb, num_nodes)
    return policy[:, 0].reshape(bs, num_nodes)


# speedup vs baseline: 1.0716x; 1.0716x over previous
import functools

import jax
import jax.numpy as jnp
from jax import lax
from jax.experimental import pallas as pl
from jax.experimental.pallas import tpu as pltpu
from jax.experimental.pallas import tpu_sc as plsc

D = 128
H = 8
DH = 16
HD = 64
WA = 128
NC = 2
NS = 16
NW = NC * NS
CH = 80
RB = 2000


def _sc_gather(KV, Q, src, dst):
    N, DKV = KV.shape
    NE = src.shape[0]
    per_w = NE // NW
    iters = per_w // CH
    pairs = (iters - 1) // 2
    mesh = plsc.VectorSubcoreMesh(core_axis_name="c", subcore_axis_name="s")

    @functools.partial(
        pl.kernel,
        out_type=[jax.ShapeDtypeStruct((NE, DKV), jnp.float32),
                  jax.ShapeDtypeStruct((NE, D), jnp.float32)],
        mesh=mesh,
        scratch_types=[
            pltpu.VMEM((per_w,), jnp.int32),
            pltpu.VMEM((per_w,), jnp.int32),
            pltpu.VMEM((CH, DKV), jnp.float32),
            pltpu.VMEM((CH, DKV), jnp.float32),
            pltpu.VMEM((CH, D), jnp.float32),
            pltpu.VMEM((CH, D), jnp.float32),
            pltpu.SemaphoreType.DMA,
            pltpu.SemaphoreType.DMA,
            pltpu.SemaphoreType.DMA,
            pltpu.SemaphoreType.DMA,
        ],
    )
    def gather_k(kv_hbm, q_hbm, src_hbm, dst_hbm, kv_out, q_out,
                 idx_s, idx_d, bkva, bkvb, bqa, bqb,
                 ska, skb, sqa, sqb):
        wid = lax.axis_index("s") * NC + lax.axis_index("c")
        base = wid * per_w
        pltpu.sync_copy(src_hbm.at[pl.ds(base, per_w)], idx_s)
        pltpu.sync_copy(dst_hbm.at[pl.ds(base, per_w)], idx_d)

        def fire(j, bkv, bq, skv, sq):
            pltpu.async_copy(kv_hbm.at[idx_s.at[pl.ds(j * CH, CH)]], bkv, skv)
            pltpu.async_copy(q_hbm.at[idx_d.at[pl.ds(j * CH, CH)]], bq, sq)

        def finish(j, bkv, bq, skv, sq):
            off = base + j * CH
            pltpu.make_async_copy(
                kv_hbm.at[idx_s.at[pl.ds(j * CH, CH)]], bkv, skv).wait()
            pltpu.make_async_copy(
                q_hbm.at[idx_d.at[pl.ds(j * CH, CH)]], bq, sq).wait()
            pltpu.sync_copy(bkv, kv_out.at[pl.ds(off, CH)])
            pltpu.sync_copy(bq, q_out.at[pl.ds(off, CH)])

        fire(0, bkva, bqa, ska, sqa)

        def body(i, carry):
            j = 2 * i
            fire(j + 1, bkvb, bqb, skb, sqb)
            finish(j, bkva, bqa, ska, sqa)
            fire(j + 2, bkva, bqa, ska, sqa)
            finish(j + 1, bkvb, bqb, skb, sqb)
            return carry

        lax.fori_loop(0, pairs, body, 0)
        finish(iters - 1, bkva, bqa, ska, sqa)

    return gather_k(KV, Q, src, dst)


def _sc_scatter(data, dst, N):
    NE, W = data.shape
    per_w = NE // NW
    iters = per_w // CH
    RC = 80
    RT = 640
    zeros_w = jnp.zeros((RC, W), jnp.float32)
    mesh = plsc.VectorSubcoreMesh(core_axis_name="c", subcore_axis_name="s")

    @functools.partial(
        pl.kernel,
        out_type=jax.ShapeDtypeStruct((NC, N, W), jnp.float32),
        mesh=mesh,
        scratch_types=[
            pltpu.VMEM((per_w,), jnp.int32),
            pltpu.VMEM((CH, W), jnp.float32),
            pltpu.VMEM((CH, W), jnp.float32),
            pltpu.VMEM((RC, W), jnp.float32),
            pltpu.VMEM_SHARED((N, W), jnp.float32),
            pltpu.SemaphoreType.DMA,
            pltpu.SemaphoreType.DMA,
            pltpu.SemaphoreType.DMA,
            pltpu.SemaphoreType.DMA,
        ],
    )
    def scatter_k(d_hbm, dst_hbm, z_hbm, out, idx, eba, ebb, ib,
                  acc, sfa, sfb, _su1, _su2):
        cid = lax.axis_index("c")
        sid = lax.axis_index("s")
        wid = sid * NC + cid
        row0 = sid * RT
        n_chunks = jnp.minimum(N - row0, RT) // RC
        base = wid * per_w

        pltpu.sync_copy(z_hbm, ib)

        def initj(j, carry):
            pltpu.sync_copy(ib, acc.at[pl.ds(row0 + j * RC, RC)])
            return carry

        lax.fori_loop(0, n_chunks, initj, 0)
        pltpu.sync_copy(dst_hbm.at[pl.ds(base, per_w)], idx)
        plsc.subcore_barrier()

        def idx_sl(j):
            return idx.at[pl.ds(j * CH, CH)]

        def fetch(j, eb, sf):
            pltpu.async_copy(d_hbm.at[pl.ds(base + j * CH, CH)], eb, sf)

        def wait_fetch(j, eb, sf):
            pltpu.make_async_copy(
                d_hbm.at[pl.ds(base + j * CH, CH)], eb, sf).wait()

        def add(j, eb):
            pltpu.sync_copy(eb, acc.at[idx_sl(j)], add=True)

        fetch(0, eba, sfa)

        def body(i, carry):
            j = 2 * i
            fetch(j + 1, ebb, sfb)
            wait_fetch(j, eba, sfa)
            add(j, eba)
            fetch(j + 2, eba, sfa)
            wait_fetch(j + 1, ebb, sfb)
            add(j + 1, ebb)
            return carry

        lax.fori_loop(0, (iters - 1) // 2, body, 0)
        wait_fetch(iters - 1, eba, sfa)
        add(iters - 1, eba)
        plsc.subcore_barrier()

        def outj(j, carry):
            r = row0 + j * RC
            pltpu.sync_copy(acc.at[pl.ds(r, RC)], ib)
            pltpu.sync_copy(ib, out.at[cid, pl.ds(r, RC)])
            return carry

        lax.fori_loop(0, n_chunks, outj, 0)

    return scatter_k(data, dst, zeros_w)


def _dot(a, b):
    return jnp.dot(a, b, preferred_element_type=jnp.float32)


def _full(shape=None):
    return pl.BlockSpec(memory_space=pltpu.ANY) if shape is None else \
        pl.BlockSpec(shape, lambda i: (0,) * len(shape))


def _rows(shape):
    return pl.BlockSpec(shape, lambda i: (i,) + (0,) * (len(shape) - 1))


def _tc_embed_qkv(h, embW, embb, qW, qb, kW, kb, vW, vb):
    N = h.shape[0]

    def body(h_ref, ew_ref, eb_ref, qw_ref, qb_ref, kw_ref, kb_ref,
             vw_ref, vb_ref, hh_ref, q_ref, kv_ref):
        hh = _dot(h_ref[...], ew_ref[...]) + eb_ref[...]
        hh_ref[...] = hh
        q_ref[...] = _dot(hh, qw_ref[...]) + qb_ref[...]
        kv_ref[...] = jnp.concatenate(
            [_dot(hh, kw_ref[...]) + kb_ref[...],
             _dot(hh, vw_ref[...]) + vb_ref[...]], axis=1)

    return pl.pallas_call(
        body,
        out_shape=[jax.ShapeDtypeStruct((N, D), jnp.float32)] * 2
        + [jax.ShapeDtypeStruct((N, 2 * D), jnp.float32)],
    )(h, embW, embb, qW, qb, kW, kb, vW, vb)


def _tc_edge_full(kvsrc, qdst, e, embW, embb, eW, eb, oeW, oeb,
                  ones_hb, ones_hb_t):
    NE = kvsrc.shape[0]
    F = e.shape[1]
    grid = NE // RB

    def body(kv_ref, qd_ref, e_ref, ew_ref, ebias_ref, pw_ref, pb_ref,
             ow_ref, ob_ref, hb_ref, hbt_ref,
             wva_ref, wvb_ref, epre_ref, st_ref):
        ee = _dot(e_ref[...], ew_ref[...]) + ebias_ref[...]
        ep = _dot(ee, pw_ref[...]) + pb_ref[...]
        score = kv_ref[:, :D] * qd_ref[...] * ep * 0.25
        ssum = _dot(score, hbt_ref[...])
        attn = jnp.exp(jnp.clip(ssum, -5.0, 5.0))
        attnb = _dot(attn, hb_ref[...])
        wv = kv_ref[:, D:] * attnb
        pad = jnp.zeros((RB, WA - HD - H), jnp.float32)
        wva_ref[...] = jnp.concatenate([wv[:, :HD], attn, pad], axis=1)
        wvb_ref[...] = jnp.concatenate([wv[:, HD:], attn, pad], axis=1)
        epre = ee + _dot(score, ow_ref[...]) + ob_ref[...]
        epre_ref[...] = epre

        @pl.when(pl.program_id(0) == 0)
        def _():
            st_ref[...] = jnp.zeros_like(st_ref)

        s = jnp.sum(epre, axis=0)
        ss = jnp.sum(epre * epre, axis=0)
        st_ref[...] += jnp.concatenate(
            [s[None], ss[None], jnp.zeros((6, D), jnp.float32)], axis=0)

    return pl.pallas_call(
        body,
        grid=(grid,),
        in_specs=[_rows((RB, 2 * D)), _rows((RB, D)), _rows((RB, F)),
                  _full((F, D)), _full((1, D)), _full((D, D)), _full((1, D)),
                  _full((D, D)), _full((1, D)), _full((H, D)), _full((D, H))],
        out_specs=[_rows((RB, WA)), _rows((RB, WA)), _rows((RB, D)),
                   _full((8, D))],
        out_shape=[jax.ShapeDtypeStruct((NE, WA), jnp.float32),
                   jax.ShapeDtypeStruct((NE, WA), jnp.float32),
                   jax.ShapeDtypeStruct((NE, D), jnp.float32),
                   jax.ShapeDtypeStruct((8, D), jnp.float32)],
    )(kvsrc, qdst, e, embW, embb, eW, eb, oeW, oeb, ones_hb, ones_hb_t)


def _tc_edge_lite(kvsrc, qdst, y, stats, g2, b2, eW, eb, n_rows,
                  ones_hb, ones_hb_t):
    NE = kvsrc.shape[0]
    grid = NE // RB
    inv_n = 1.0 / float(n_rows)

    def body(kv_ref, qd_ref, y_ref, st_ref, g_ref, b_ref, pw_ref, pb_ref,
             hb_ref, hbt_ref, wva_ref, wvb_ref):
        mean = st_ref[0:1, :] * inv_n
        var = st_ref[1:2, :] * inv_n - mean * mean
        inv = lax.rsqrt(var + 1e-5)
        x = (y_ref[...] - mean) * inv * g_ref[...] + b_ref[...]
        ep = _dot(x, pw_ref[...]) + pb_ref[...]
        score = kv_ref[:, :D] * qd_ref[...] * ep * 0.25
        attn = jnp.exp(jnp.clip(_dot(score, hbt_ref[...]), -5.0, 5.0))
        attnb = _dot(attn, hb_ref[...])
        wv = kv_ref[:, D:] * attnb
        pad = jnp.zeros((RB, WA - HD - H), jnp.float32)
        wva_ref[...] = jnp.concatenate([wv[:, :HD], attn, pad], axis=1)
        wvb_ref[...] = jnp.concatenate([wv[:, HD:], attn, pad], axis=1)

    return pl.pallas_call(
        body,
        grid=(grid,),
        in_specs=[_rows((RB, 2 * D))] + [_rows((RB, D))] * 2
        + [_full((8, D)), _full((1, D)), _full((1, D)),
           _full((D, D)), _full((1, D)), _full((H, D)), _full((D, H))],
        out_specs=[_rows((RB, WA)), _rows((RB, WA))],
        out_shape=[jax.ShapeDtypeStruct((NE, WA), jnp.float32),
                   jax.ShapeDtypeStruct((NE, WA), jnp.float32)],
    )(kvsrc, qdst, y, stats, g2, b2, eW, eb, ones_hb, ones_hb_t)


def _tc_edge_ffn(epre, stats, g1, b1, w1, bb1, w2, bb2, n_rows):
    NE = epre.shape[0]
    grid = NE // RB
    inv_n = 1.0 / float(n_rows)

    def body(ep_ref, st_ref, g_ref, b_ref, w1_ref, b1_ref, w2_ref, b2_ref,
             y_ref, sy_ref):
        mean = st_ref[0:1, :] * inv_n
        var = st_ref[1:2, :] * inv_n - mean * mean
        inv = lax.rsqrt(var + 1e-5)
        x = (ep_ref[...] - mean) * inv * g_ref[...] + b_ref[...]
        hmid = jnp.maximum(_dot(x, w1_ref[...]) + b1_ref[...], 0.0)
        y = x + _dot(hmid, w2_ref[...]) + b2_ref[...]
        y_ref[...] = y

        @pl.when(pl.program_id(0) == 0)
        def _():
            sy_ref[...] = jnp.zeros_like(sy_ref)

        s = jnp.sum(y, axis=0)
        ss = jnp.sum(y * y, axis=0)
        sy_ref[...] += jnp.concatenate(
            [s[None], ss[None], jnp.zeros((6, D), jnp.float32)], axis=0)

    return pl.pallas_call(
        body,
        grid=(grid,),
        in_specs=[_rows((RB, D)), _full((8, D)), _full((1, D)), _full((1, D)),
                  _full((D, 2 * D)), _full((1, 2 * D)),
                  _full((2 * D, D)), _full((1, D))],
        out_specs=[_rows((RB, D)), _full((8, D))],
        out_shape=[jax.ShapeDtypeStruct((NE, D), jnp.float32),
                   jax.ShapeDtypeStruct((8, D), jnp.float32)],
    )(epre, stats, g1, b1, w1, bb1, w2, bb2)


def _node_update(hpa, hpb, hh_in, lp, ones_hb):
    a = hpa[0] + hpa[1]
    b = hpb[0] + hpb[1]
    wv = jnp.concatenate([a[:, :HD], b[:, :HD]], axis=1)
    z = a[:, HD:HD + H]
    r = 1.0 / (z + 1e-6)
    h_att = wv * _dot(r, ones_hb)
    h_new = _dot(h_att, lp['oh_w']) + lp['oh_b'] + hh_in
    m = jnp.mean(h_new, axis=0, keepdims=True)
    v = jnp.mean(h_new * h_new, axis=0, keepdims=True) - m * m
    h_new = (h_new - m) * lax.rsqrt(v + 1e-5) * lp['g1'] + lp['b1']
    h2 = _dot(jnp.maximum(_dot(h_new, lp['f1w']) + lp['f1b'], 0.0),
              lp['f2w']) + lp['f2b']
    h_new = h_new + h2
    m = jnp.mean(h_new, axis=0, keepdims=True)
    v = jnp.mean(h_new * h_new, axis=0, keepdims=True) - m * m
    return (h_new - m) * lax.rsqrt(v + 1e-5) * lp['g2'] + lp['b2']


def _tc_node_mid(hpart, zpart, hh_in, lw, qW, qb, kW, kb, vW, vb, ones_hb):
    N = hh_in.shape[0]

    def body(hp_ref, zp_ref, hh_ref, ow_ref, ob_ref, g1_ref, b1_ref,
             f1w_ref, f1b_ref, f2w_ref, f2b_ref, g2_ref, b2_ref,
             qw_ref, qb_ref, kw_ref, kb_ref, vw_ref, vb_ref, hb_ref,
             hh_out, q_ref, kv_ref):
        lp = dict(oh_w=ow_ref[...], oh_b=ob_ref[...], g1=g1_ref[...],
                  b1=b1_ref[...], f1w=f1w_ref[...], f1b=f1b_ref[...],
                  f2w=f2w_ref[...], f2b=f2b_ref[...], g2=g2_ref[...],
                  b2=b2_ref[...])
        hh1 = _node_update(hp_ref[...], zp_ref[...], hh_ref[...], lp,
                           hb_ref[...])
        hh_out[...] = hh1
        q_ref[...] = _dot(hh1, qw_ref[...]) + qb_ref[...]
        kv_ref[...] = jnp.concatenate(
            [_dot(hh1, kw_ref[...]) + kb_ref[...],
             _dot(hh1, vw_ref[...]) + vb_ref[...]], axis=1)

    return pl.pallas_call(
        body,
        out_shape=[jax.ShapeDtypeStruct((N, D), jnp.float32)] * 2
        + [jax.ShapeDtypeStruct((N, 2 * D), jnp.float32)],
    )(hpart, zpart, hh_in, lw['oh_w'], lw['oh_b'], lw['g1'], lw['b1'],
      lw['f1w'], lw['f1b'], lw['f2w'], lw['f2b'], lw['g2'], lw['b2'],
      qW, qb, kW, kb, vW, vb, ones_hb)


def _tc_node_head(hpart, zpart, hh_in, lw, w_top, w_bot, mlp1b, mlp2w,
                  mlp2b, vid, ones_hb, num_nodes):
    N = hh_in.shape[0]
    bs = vid.shape[0]

    def body(hp_ref, zp_ref, hh_ref, ow_ref, ob_ref, g1_ref, b1_ref,
             f1w_ref, f1b_ref, f2w_ref, f2b_ref, g2_ref, b2_ref,
             wt_ref, wb_ref, m1b_ref, m2w_ref, m2b_ref, hb_ref, vid_ref,
             out_ref, hh2_ref):
        lp = dict(oh_w=ow_ref[...], oh_b=ob_ref[...], g1=g1_ref[...],
                  b1=b1_ref[...], f1w=f1w_ref[...], f1b=f1b_ref[...],
                  f2w=f2w_ref[...], f2b=f2b_ref[...], g2=g2_ref[...],
                  b2=b2_ref[...])
        hh2 = _node_update(hp_ref[...], zp_ref[...], hh_ref[...], lp,
                           hb_ref[...])
        hh2_ref[...] = hh2
        for b in range(bs):
            row = hh2_ref[pl.ds(vid_ref[b], 1), :]
            cb = _dot(row, wt_ref[...]) + m1b_ref[...]
            blk = hh2_ref[pl.ds(b * num_nodes, num_nodes), :]
            t = jnp.maximum(_dot(blk, wb_ref[...]) + cb, 0.0)
            out_ref[pl.ds(b * num_nodes, num_nodes), :] = (
                _dot(t, m2w_ref[...]) + m2b_ref[...])

    return pl.pallas_call(
        body,
        in_specs=[pl.BlockSpec(memory_space=pltpu.VMEM)] * 19
        + [pl.BlockSpec(memory_space=pltpu.SMEM)],
        out_specs=pl.BlockSpec(memory_space=pltpu.VMEM),
        out_shape=jax.ShapeDtypeStruct((N, 1), jnp.float32),
        scratch_shapes=[pltpu.VMEM((N, D), jnp.float32)],
    )(hpart, zpart, hh_in, lw['oh_w'], lw['oh_b'], lw['g1'], lw['b1'],
      lw['f1w'], lw['f1b'], lw['f2w'], lw['f2b'], lw['g2'], lw['b2'],
      w_top, w_bot, mlp1b, mlp2w, mlp2b, ones_hb, vid)


def _row(x):
    return x.reshape(1, -1)


def _layer_w(lp):
    return dict(oh_w=lp['Oh_W'], oh_b=_row(lp['Oh_b']),
                g1=_row(lp['bn1h_g']), b1=_row(lp['bn1h_b']),
                f1w=lp['ffh1_W'], f1b=_row(lp['ffh1_b']),
                f2w=lp['ffh2_W'], f2b=_row(lp['ffh2_b']),
                g2=_row(lp['bn2h_g']), b2=_row(lp['bn2h_b']))


def kernel(h, e, params, edge_index, vehicle_node_id, batch_size):
    src = edge_index[0]
    dst = edge_index[1]
    N = h.shape[0]
    NE = src.shape[0]
    bs = vehicle_node_id.shape[0]
    num_nodes = N // bs
    L1, L2 = params['layers']

    ones_hb = jnp.repeat(jnp.eye(H, dtype=jnp.float32), DH, axis=1)
    ones_hb_t = ones_hb.T

    hh0, q1, kv1 = _tc_embed_qkv(
        h, params['emb_h_W'], _row(params['emb_h_b']),
        L1['Q_W'], _row(L1['Q_b']), L1['K_W'], _row(L1['K_b']),
        L1['V_W'], _row(L1['V_b']))

    kvsrc1, qdst1 = _sc_gather(kv1, q1, src, dst)
    wv1a, wv1b, epre1, st1 = _tc_edge_full(
        kvsrc1, qdst1, e, params['emb_e_W'], _row(params['emb_e_b']),
        L1['E_W'], _row(L1['E_b']), L1['Oe_W'], _row(L1['Oe_b']),
        ones_hb, ones_hb_t)
    hpa1 = _sc_scatter(wv1a, dst, N)
    hpb1 = _sc_scatter(wv1b, dst, N)
    hh1, q2, kv2 = _tc_node_mid(
        hpa1, hpb1, hh0, _layer_w(L1),
        L2['Q_W'], _row(L2['Q_b']), L2['K_W'], _row(L2['K_b']),
        L2['V_W'], _row(L2['V_b']), ones_hb)
    y1, sty1 = _tc_edge_ffn(
        epre1, st1, _row(L1['bn1e_g']), _row(L1['bn1e_b']),
        L1['ffe1_W'], _row(L1['ffe1_b']), L1['ffe2_W'], _row(L1['ffe2_b']),
        NE)

    kvsrc2, qdst2 = _sc_gather(kv2, q2, src, dst)
    wv2a, wv2b = _tc_edge_lite(
        kvsrc2, qdst2, y1, sty1, _row(L1['bn2e_g']), _row(L1['bn2e_b']),
        L2['E_W'], _row(L2['E_b']), NE, ones_hb, ones_hb_t)
    hpa2 = _sc_scatter(wv2a, dst, N)
    hpb2 = _sc_scatter(wv2b, dst, N)

    vid = (vehicle_node_id.astype(jnp.int32)
           + jnp.arange(bs, dtype=jnp.int32) * num_nodes
           + (jnp.asarray(batch_size, jnp.int32) - jnp.int32(bs)))
    policy = _tc_node_head(
        hpa2, hpb2, hh1, _layer_w(L2),
        params['mlp1_W'][:D], params['mlp1_W'][D:], _row(params['mlp1_b']),
        params['mlp2_W'], _row(params['mlp2_b']), vid, ones_hb, num_nodes)
    return policy[:, 0].reshape(bs, num_nodes)


# speedup vs baseline: 1.0944x; 1.0213x over previous
import functools

import jax
import jax.numpy as jnp
from jax import lax
from jax.experimental import pallas as pl
from jax.experimental.pallas import tpu as pltpu
from jax.experimental.pallas import tpu_sc as plsc

D = 128
H = 8
DH = 16
HD = 64
WA = 128
NC = 2
NS = 16
NW = NC * NS
CH = 80
RB = 2000


def _sc_gather(KV, Q, src, dst):
    N, DKV = KV.shape
    NE = src.shape[0]
    per_w = NE // NW
    CHG = 128
    full = per_w // CHG
    tail = per_w - full * CHG
    mesh = plsc.VectorSubcoreMesh(core_axis_name="c", subcore_axis_name="s")

    @functools.partial(
        pl.kernel,
        out_type=[jax.ShapeDtypeStruct((NE, DKV), jnp.float32),
                  jax.ShapeDtypeStruct((NE, D), jnp.float32)],
        mesh=mesh,
        scratch_types=[
            pltpu.VMEM((per_w,), jnp.int32),
            pltpu.VMEM((per_w,), jnp.int32),
            pltpu.VMEM((CHG, DKV), jnp.float32),
            pltpu.VMEM((CHG, DKV), jnp.float32),
            pltpu.VMEM((CHG, D), jnp.float32),
            pltpu.VMEM((CHG, D), jnp.float32),
            pltpu.SemaphoreType.DMA,
            pltpu.SemaphoreType.DMA,
            pltpu.SemaphoreType.DMA,
            pltpu.SemaphoreType.DMA,
        ],
    )
    def gather_k(kv_hbm, q_hbm, src_hbm, dst_hbm, kv_out, q_out,
                 idx_s, idx_d, bkva, bkvb, bqa, bqb,
                 ska, skb, sqa, sqb):
        wid = lax.axis_index("s") * NC + lax.axis_index("c")
        base = wid * per_w
        pltpu.sync_copy(src_hbm.at[pl.ds(base, per_w)], idx_s)
        pltpu.sync_copy(dst_hbm.at[pl.ds(base, per_w)], idx_d)

        def fire(j, bkv, bq, skv, sq):
            o = j * CHG
            pltpu.async_copy(kv_hbm.at[idx_s.at[pl.ds(o, CHG)]], bkv, skv)
            pltpu.async_copy(q_hbm.at[idx_d.at[pl.ds(o, CHG)]], bq, sq)

        def finish(j, bkv, bq, skv, sq):
            o = j * CHG
            pltpu.make_async_copy(
                kv_hbm.at[idx_s.at[pl.ds(o, CHG)]], bkv, skv).wait()
            pltpu.make_async_copy(
                q_hbm.at[idx_d.at[pl.ds(o, CHG)]], bq, sq).wait()
            pltpu.sync_copy(bkv, kv_out.at[pl.ds(base + o, CHG)])
            pltpu.sync_copy(bq, q_out.at[pl.ds(base + o, CHG)])

        fire(0, bkva, bqa, ska, sqa)

        def body(i, carry):
            j = 2 * i
            fire(j + 1, bkvb, bqb, skb, sqb)
            finish(j, bkva, bqa, ska, sqa)
            fire(j + 2, bkva, bqa, ska, sqa)
            finish(j + 1, bkvb, bqb, skb, sqb)
            return carry

        lax.fori_loop(0, (full - 2) // 2, body, 0)
        fire(full - 1, bkvb, bqb, skb, sqb)
        finish(full - 2, bkva, bqa, ska, sqa)
        finish(full - 1, bkvb, bqb, skb, sqb)
        if tail:
            to = full * CHG
            pltpu.async_copy(kv_hbm.at[idx_s.at[pl.ds(to, tail)]],
                             bkva.at[pl.ds(0, tail)], ska)
            pltpu.async_copy(q_hbm.at[idx_d.at[pl.ds(to, tail)]],
                             bqa.at[pl.ds(0, tail)], sqa)
            pltpu.make_async_copy(kv_hbm.at[idx_s.at[pl.ds(to, tail)]],
                                  bkva.at[pl.ds(0, tail)], ska).wait()
            pltpu.make_async_copy(q_hbm.at[idx_d.at[pl.ds(to, tail)]],
                                  bqa.at[pl.ds(0, tail)], sqa).wait()
            pltpu.sync_copy(bkva.at[pl.ds(0, tail)],
                            kv_out.at[pl.ds(base + to, tail)])
            pltpu.sync_copy(bqa.at[pl.ds(0, tail)],
                            q_out.at[pl.ds(base + to, tail)])

    return gather_k(KV, Q, src, dst)


def _sc_scatter(data, dst, N):
    NE, W = data.shape
    per_w = NE // NW
    CHG = 128
    full = per_w // CHG
    tail = per_w - full * CHG
    RC = 80
    RT = 640
    zeros_w = jnp.zeros((RC, W), jnp.float32)
    mesh = plsc.VectorSubcoreMesh(core_axis_name="c", subcore_axis_name="s")

    @functools.partial(
        pl.kernel,
        out_type=jax.ShapeDtypeStruct((NC, N, W), jnp.float32),
        mesh=mesh,
        scratch_types=[
            pltpu.VMEM((per_w,), jnp.int32),
            pltpu.VMEM((CHG, W), jnp.float32),
            pltpu.VMEM((CHG, W), jnp.float32),
            pltpu.VMEM_SHARED((N, W), jnp.float32),
            pltpu.SemaphoreType.DMA,
            pltpu.SemaphoreType.DMA,
        ],
    )
    def scatter_k(d_hbm, dst_hbm, z_hbm, out, idx, eba, ebb,
                  acc, sfa, sfb):
        cid = lax.axis_index("c")
        sid = lax.axis_index("s")
        wid = sid * NC + cid
        row0 = sid * RT
        n_chunks = jnp.minimum(N - row0, RT) // RC
        base = wid * per_w

        ib = eba.at[pl.ds(0, RC)]
        pltpu.sync_copy(z_hbm, ib)

        def initj(j, carry):
            pltpu.sync_copy(ib, acc.at[pl.ds(row0 + j * RC, RC)])
            return carry

        lax.fori_loop(0, n_chunks, initj, 0)
        pltpu.sync_copy(dst_hbm.at[pl.ds(base, per_w)], idx)
        plsc.subcore_barrier()

        def fetch(j, eb, sf):
            pltpu.async_copy(d_hbm.at[pl.ds(base + j * CHG, CHG)], eb, sf)

        def wait_fetch(j, eb, sf):
            pltpu.make_async_copy(
                d_hbm.at[pl.ds(base + j * CHG, CHG)], eb, sf).wait()

        def add(j, eb):
            pltpu.sync_copy(eb, acc.at[idx.at[pl.ds(j * CHG, CHG)]],
                            add=True)

        fetch(0, eba, sfa)

        def body(i, carry):
            j = 2 * i
            fetch(j + 1, ebb, sfb)
            wait_fetch(j, eba, sfa)
            add(j, eba)
            fetch(j + 2, eba, sfa)
            wait_fetch(j + 1, ebb, sfb)
            add(j + 1, ebb)
            return carry

        lax.fori_loop(0, (full - 2) // 2, body, 0)
        fetch(full - 1, ebb, sfb)
        wait_fetch(full - 2, eba, sfa)
        add(full - 2, eba)
        wait_fetch(full - 1, ebb, sfb)
        add(full - 1, ebb)
        if tail:
            to = full * CHG
            pltpu.async_copy(d_hbm.at[pl.ds(base + to, tail)],
                             eba.at[pl.ds(0, tail)], sfa)
            pltpu.make_async_copy(d_hbm.at[pl.ds(base + to, tail)],
                                  eba.at[pl.ds(0, tail)], sfa).wait()
            pltpu.sync_copy(eba.at[pl.ds(0, tail)],
                            acc.at[idx.at[pl.ds(to, tail)]], add=True)
        plsc.subcore_barrier()

        def outj(j, carry):
            r = row0 + j * RC
            pltpu.sync_copy(acc.at[pl.ds(r, RC)], ib)
            pltpu.sync_copy(ib, out.at[cid, pl.ds(r, RC)])
            return carry

        lax.fori_loop(0, n_chunks, outj, 0)

    return scatter_k(data, dst, zeros_w)


def _dot(a, b):
    return jnp.dot(a, b, preferred_element_type=jnp.float32)


def _full(shape=None):
    return pl.BlockSpec(memory_space=pltpu.ANY) if shape is None else \
        pl.BlockSpec(shape, lambda i: (0,) * len(shape))


def _rows(shape):
    return pl.BlockSpec(shape, lambda i: (i,) + (0,) * (len(shape) - 1))


def _tc_embed_qkv(h, embW, embb, qW, qb, kW, kb, vW, vb):
    N = h.shape[0]

    def body(h_ref, ew_ref, eb_ref, qw_ref, qb_ref, kw_ref, kb_ref,
             vw_ref, vb_ref, hh_ref, q_ref, kv_ref):
        hh = _dot(h_ref[...], ew_ref[...]) + eb_ref[...]
        hh_ref[...] = hh
        q_ref[...] = _dot(hh, qw_ref[...]) + qb_ref[...]
        kv_ref[...] = jnp.concatenate(
            [_dot(hh, kw_ref[...]) + kb_ref[...],
             _dot(hh, vw_ref[...]) + vb_ref[...]], axis=1)

    return pl.pallas_call(
        body,
        out_shape=[jax.ShapeDtypeStruct((N, D), jnp.float32)] * 2
        + [jax.ShapeDtypeStruct((N, 2 * D), jnp.float32)],
    )(h, embW, embb, qW, qb, kW, kb, vW, vb)


def _tc_edge_full(kvsrc, qdst, e, embW, embb, eW, eb, oeW, oeb,
                  ones_hb, ones_hb_t):
    NE = kvsrc.shape[0]
    F = e.shape[1]
    grid = NE // RB

    def body(kv_ref, qd_ref, e_ref, ew_ref, ebias_ref, pw_ref, pb_ref,
             ow_ref, ob_ref, hb_ref, hbt_ref,
             wva_ref, wvb_ref, epre_ref, st_ref):
        ee = _dot(e_ref[...], ew_ref[...]) + ebias_ref[...]
        ep = _dot(ee, pw_ref[...]) + pb_ref[...]
        score = kv_ref[:, :D] * qd_ref[...] * ep * 0.25
        ssum = _dot(score, hbt_ref[...])
        attn = jnp.exp(jnp.clip(ssum, -5.0, 5.0))
        attnb = _dot(attn, hb_ref[...])
        wv = kv_ref[:, D:] * attnb
        pad = jnp.zeros((RB, WA - HD - H), jnp.float32)
        wva_ref[...] = jnp.concatenate([wv[:, :HD], attn, pad], axis=1)
        wvb_ref[...] = jnp.concatenate([wv[:, HD:], attn, pad], axis=1)
        epre = ee + _dot(score, ow_ref[...]) + ob_ref[...]
        epre_ref[...] = epre

        @pl.when(pl.program_id(0) == 0)
        def _():
            st_ref[...] = jnp.zeros_like(st_ref)

        s = jnp.sum(epre, axis=0)
        ss = jnp.sum(epre * epre, axis=0)
        st_ref[...] += jnp.concatenate(
            [s[None], ss[None], jnp.zeros((6, D), jnp.float32)], axis=0)

    return pl.pallas_call(
        body,
        grid=(grid,),
        in_specs=[_rows((RB, 2 * D)), _rows((RB, D)), _rows((RB, F)),
                  _full((F, D)), _full((1, D)), _full((D, D)), _full((1, D)),
                  _full((D, D)), _full((1, D)), _full((H, D)), _full((D, H))],
        out_specs=[_rows((RB, WA)), _rows((RB, WA)), _rows((RB, D)),
                   _full((8, D))],
        out_shape=[jax.ShapeDtypeStruct((NE, WA), jnp.float32),
                   jax.ShapeDtypeStruct((NE, WA), jnp.float32),
                   jax.ShapeDtypeStruct((NE, D), jnp.float32),
                   jax.ShapeDtypeStruct((8, D), jnp.float32)],
    )(kvsrc, qdst, e, embW, embb, eW, eb, oeW, oeb, ones_hb, ones_hb_t)


def _tc_edge_lite(kvsrc, qdst, y, stats, g2, b2, eW, eb, n_rows,
                  ones_hb, ones_hb_t):
    NE = kvsrc.shape[0]
    grid = NE // RB
    inv_n = 1.0 / float(n_rows)

    def body(kv_ref, qd_ref, y_ref, st_ref, g_ref, b_ref, pw_ref, pb_ref,
             hb_ref, hbt_ref, wva_ref, wvb_ref):
        mean = st_ref[0:1, :] * inv_n
        var = st_ref[1:2, :] * inv_n - mean * mean
        inv = lax.rsqrt(var + 1e-5)
        x = (y_ref[...] - mean) * inv * g_ref[...] + b_ref[...]
        ep = _dot(x, pw_ref[...]) + pb_ref[...]
        score = kv_ref[:, :D] * qd_ref[...] * ep * 0.25
        attn = jnp.exp(jnp.clip(_dot(score, hbt_ref[...]), -5.0, 5.0))
        attnb = _dot(attn, hb_ref[...])
        wv = kv_ref[:, D:] * attnb
        pad = jnp.zeros((RB, WA - HD - H), jnp.float32)
        wva_ref[...] = jnp.concatenate([wv[:, :HD], attn, pad], axis=1)
        wvb_ref[...] = jnp.concatenate([wv[:, HD:], attn, pad], axis=1)

    return pl.pallas_call(
        body,
        grid=(grid,),
        in_specs=[_rows((RB, 2 * D))] + [_rows((RB, D))] * 2
        + [_full((8, D)), _full((1, D)), _full((1, D)),
           _full((D, D)), _full((1, D)), _full((H, D)), _full((D, H))],
        out_specs=[_rows((RB, WA)), _rows((RB, WA))],
        out_shape=[jax.ShapeDtypeStruct((NE, WA), jnp.float32),
                   jax.ShapeDtypeStruct((NE, WA), jnp.float32)],
    )(kvsrc, qdst, y, stats, g2, b2, eW, eb, ones_hb, ones_hb_t)


def _tc_edge_ffn(epre, stats, g1, b1, w1, bb1, w2, bb2, n_rows):
    NE = epre.shape[0]
    grid = NE // RB
    inv_n = 1.0 / float(n_rows)

    def body(ep_ref, st_ref, g_ref, b_ref, w1_ref, b1_ref, w2_ref, b2_ref,
             y_ref, sy_ref):
        mean = st_ref[0:1, :] * inv_n
        var = st_ref[1:2, :] * inv_n - mean * mean
        inv = lax.rsqrt(var + 1e-5)
        x = (ep_ref[...] - mean) * inv * g_ref[...] + b_ref[...]
        hmid = jnp.maximum(_dot(x, w1_ref[...]) + b1_ref[...], 0.0)
        y = x + _dot(hmid, w2_ref[...]) + b2_ref[...]
        y_ref[...] = y

        @pl.when(pl.program_id(0) == 0)
        def _():
            sy_ref[...] = jnp.zeros_like(sy_ref)

        s = jnp.sum(y, axis=0)
        ss = jnp.sum(y * y, axis=0)
        sy_ref[...] += jnp.concatenate(
            [s[None], ss[None], jnp.zeros((6, D), jnp.float32)], axis=0)

    return pl.pallas_call(
        body,
        grid=(grid,),
        in_specs=[_rows((RB, D)), _full((8, D)), _full((1, D)), _full((1, D)),
                  _full((D, 2 * D)), _full((1, 2 * D)),
                  _full((2 * D, D)), _full((1, D))],
        out_specs=[_rows((RB, D)), _full((8, D))],
        out_shape=[jax.ShapeDtypeStruct((NE, D), jnp.float32),
                   jax.ShapeDtypeStruct((8, D), jnp.float32)],
    )(epre, stats, g1, b1, w1, bb1, w2, bb2)


def _node_update(hpa, hpb, hh_in, lp, ones_hb):
    a = hpa[0] + hpa[1]
    b = hpb[0] + hpb[1]
    wv = jnp.concatenate([a[:, :HD], b[:, :HD]], axis=1)
    z = a[:, HD:HD + H]
    r = 1.0 / (z + 1e-6)
    h_att = wv * _dot(r, ones_hb)
    h_new = _dot(h_att, lp['oh_w']) + lp['oh_b'] + hh_in
    m = jnp.mean(h_new, axis=0, keepdims=True)
    v = jnp.mean(h_new * h_new, axis=0, keepdims=True) - m * m
    h_new = (h_new - m) * lax.rsqrt(v + 1e-5) * lp['g1'] + lp['b1']
    h2 = _dot(jnp.maximum(_dot(h_new, lp['f1w']) + lp['f1b'], 0.0),
              lp['f2w']) + lp['f2b']
    h_new = h_new + h2
    m = jnp.mean(h_new, axis=0, keepdims=True)
    v = jnp.mean(h_new * h_new, axis=0, keepdims=True) - m * m
    return (h_new - m) * lax.rsqrt(v + 1e-5) * lp['g2'] + lp['b2']


def _tc_node_mid(hpart, zpart, hh_in, lw, qW, qb, kW, kb, vW, vb, ones_hb):
    N = hh_in.shape[0]

    def body(hp_ref, zp_ref, hh_ref, ow_ref, ob_ref, g1_ref, b1_ref,
             f1w_ref, f1b_ref, f2w_ref, f2b_ref, g2_ref, b2_ref,
             qw_ref, qb_ref, kw_ref, kb_ref, vw_ref, vb_ref, hb_ref,
             hh_out, q_ref, kv_ref):
        lp = dict(oh_w=ow_ref[...], oh_b=ob_ref[...], g1=g1_ref[...],
                  b1=b1_ref[...], f1w=f1w_ref[...], f1b=f1b_ref[...],
                  f2w=f2w_ref[...], f2b=f2b_ref[...], g2=g2_ref[...],
                  b2=b2_ref[...])
        hh1 = _node_update(hp_ref[...], zp_ref[...], hh_ref[...], lp,
                           hb_ref[...])
        hh_out[...] = hh1
        q_ref[...] = _dot(hh1, qw_ref[...]) + qb_ref[...]
        kv_ref[...] = jnp.concatenate(
            [_dot(hh1, kw_ref[...]) + kb_ref[...],
             _dot(hh1, vw_ref[...]) + vb_ref[...]], axis=1)

    return pl.pallas_call(
        body,
        out_shape=[jax.ShapeDtypeStruct((N, D), jnp.float32)] * 2
        + [jax.ShapeDtypeStruct((N, 2 * D), jnp.float32)],
    )(hpart, zpart, hh_in, lw['oh_w'], lw['oh_b'], lw['g1'], lw['b1'],
      lw['f1w'], lw['f1b'], lw['f2w'], lw['f2b'], lw['g2'], lw['b2'],
      qW, qb, kW, kb, vW, vb, ones_hb)


def _tc_node_head(hpart, zpart, hh_in, lw, w_top, w_bot, mlp1b, mlp2w,
                  mlp2b, vid, ones_hb, num_nodes):
    N = hh_in.shape[0]
    bs = vid.shape[0]

    def body(hp_ref, zp_ref, hh_ref, ow_ref, ob_ref, g1_ref, b1_ref,
             f1w_ref, f1b_ref, f2w_ref, f2b_ref, g2_ref, b2_ref,
             wt_ref, wb_ref, m1b_ref, m2w_ref, m2b_ref, hb_ref, vid_ref,
             out_ref, hh2_ref):
        lp = dict(oh_w=ow_ref[...], oh_b=ob_ref[...], g1=g1_ref[...],
                  b1=b1_ref[...], f1w=f1w_ref[...], f1b=f1b_ref[...],
                  f2w=f2w_ref[...], f2b=f2b_ref[...], g2=g2_ref[...],
                  b2=b2_ref[...])
        hh2 = _node_update(hp_ref[...], zp_ref[...], hh_ref[...], lp,
                           hb_ref[...])
        hh2_ref[...] = hh2
        for b in range(bs):
            row = hh2_ref[pl.ds(vid_ref[b], 1), :]
            cb = _dot(row, wt_ref[...]) + m1b_ref[...]
            blk = hh2_ref[pl.ds(b * num_nodes, num_nodes), :]
            t = jnp.maximum(_dot(blk, wb_ref[...]) + cb, 0.0)
            out_ref[pl.ds(b * num_nodes, num_nodes), :] = (
                _dot(t, m2w_ref[...]) + m2b_ref[...])

    return pl.pallas_call(
        body,
        in_specs=[pl.BlockSpec(memory_space=pltpu.VMEM)] * 19
        + [pl.BlockSpec(memory_space=pltpu.SMEM)],
        out_specs=pl.BlockSpec(memory_space=pltpu.VMEM),
        out_shape=jax.ShapeDtypeStruct((N, 1), jnp.float32),
        scratch_shapes=[pltpu.VMEM((N, D), jnp.float32)],
    )(hpart, zpart, hh_in, lw['oh_w'], lw['oh_b'], lw['g1'], lw['b1'],
      lw['f1w'], lw['f1b'], lw['f2w'], lw['f2b'], lw['g2'], lw['b2'],
      w_top, w_bot, mlp1b, mlp2w, mlp2b, ones_hb, vid)


def _row(x):
    return x.reshape(1, -1)


def _layer_w(lp):
    return dict(oh_w=lp['Oh_W'], oh_b=_row(lp['Oh_b']),
                g1=_row(lp['bn1h_g']), b1=_row(lp['bn1h_b']),
                f1w=lp['ffh1_W'], f1b=_row(lp['ffh1_b']),
                f2w=lp['ffh2_W'], f2b=_row(lp['ffh2_b']),
                g2=_row(lp['bn2h_g']), b2=_row(lp['bn2h_b']))


def kernel(h, e, params, edge_index, vehicle_node_id, batch_size):
    src = edge_index[0]
    dst = edge_index[1]
    N = h.shape[0]
    NE = src.shape[0]
    bs = vehicle_node_id.shape[0]
    num_nodes = N // bs
    L1, L2 = params['layers']

    ones_hb = jnp.repeat(jnp.eye(H, dtype=jnp.float32), DH, axis=1)
    ones_hb_t = ones_hb.T

    hh0, q1, kv1 = _tc_embed_qkv(
        h, params['emb_h_W'], _row(params['emb_h_b']),
        L1['Q_W'], _row(L1['Q_b']), L1['K_W'], _row(L1['K_b']),
        L1['V_W'], _row(L1['V_b']))

    kvsrc1, qdst1 = _sc_gather(kv1, q1, src, dst)
    wv1a, wv1b, epre1, st1 = _tc_edge_full(
        kvsrc1, qdst1, e, params['emb_e_W'], _row(params['emb_e_b']),
        L1['E_W'], _row(L1['E_b']), L1['Oe_W'], _row(L1['Oe_b']),
        ones_hb, ones_hb_t)
    hpa1 = _sc_scatter(wv1a, dst, N)
    hpb1 = _sc_scatter(wv1b, dst, N)
    hh1, q2, kv2 = _tc_node_mid(
        hpa1, hpb1, hh0, _layer_w(L1),
        L2['Q_W'], _row(L2['Q_b']), L2['K_W'], _row(L2['K_b']),
        L2['V_W'], _row(L2['V_b']), ones_hb)
    y1, sty1 = _tc_edge_ffn(
        epre1, st1, _row(L1['bn1e_g']), _row(L1['bn1e_b']),
        L1['ffe1_W'], _row(L1['ffe1_b']), L1['ffe2_W'], _row(L1['ffe2_b']),
        NE)

    kvsrc2, qdst2 = _sc_gather(kv2, q2, src, dst)
    wv2a, wv2b = _tc_edge_lite(
        kvsrc2, qdst2, y1, sty1, _row(L1['bn2e_g']), _row(L1['bn2e_b']),
        L2['E_W'], _row(L2['E_b']), NE, ones_hb, ones_hb_t)
    hpa2 = _sc_scatter(wv2a, dst, N)
    hpb2 = _sc_scatter(wv2b, dst, N)

    vid = (vehicle_node_id.astype(jnp.int32)
           + jnp.arange(bs, dtype=jnp.int32) * num_nodes
           + (jnp.asarray(batch_size, jnp.int32) - jnp.int32(bs)))
    policy = _tc_node_head(
        hpa2, hpb2, hh1, _layer_w(L2),
        params['mlp1_W'][:D], params['mlp1_W'][D:], _row(params['mlp1_b']),
        params['mlp2_W'], _row(params['mlp2_b']), vid, ones_hb, num_nodes)
    return policy[:, 0].reshape(bs, num_nodes)


# speedup vs baseline: 1.2611x; 1.1524x over previous
import functools

import jax
import jax.numpy as jnp
from jax import lax
from jax.experimental import pallas as pl
from jax.experimental.pallas import tpu as pltpu
from jax.experimental.pallas import tpu_sc as plsc

D = 128
H = 8
DH = 16
HD = 64
WA = 128
NC = 2
NS = 16
NW = NC * NS
CH = 80
RB = 2000


def _sc_gather(KV, Q, src, dst):
    N, DKV = KV.shape
    NE = src.shape[0]
    per_w = NE // NW
    CHG = 128
    full = per_w // CHG
    tail = per_w - full * CHG
    mesh = plsc.VectorSubcoreMesh(core_axis_name="c", subcore_axis_name="s")

    @functools.partial(
        pl.kernel,
        out_type=[jax.ShapeDtypeStruct((NE, DKV), jnp.float32),
                  jax.ShapeDtypeStruct((NE, D), jnp.float32)],
        mesh=mesh,
        scratch_types=[
            pltpu.VMEM((per_w,), jnp.int32),
            pltpu.VMEM((per_w,), jnp.int32),
            pltpu.VMEM((CHG, DKV), jnp.float32),
            pltpu.VMEM((CHG, DKV), jnp.float32),
            pltpu.VMEM((CHG, D), jnp.float32),
            pltpu.VMEM((CHG, D), jnp.float32),
            pltpu.SemaphoreType.DMA,
            pltpu.SemaphoreType.DMA,
            pltpu.SemaphoreType.DMA,
            pltpu.SemaphoreType.DMA,
        ],
    )
    def gather_k(kv_hbm, q_hbm, src_hbm, dst_hbm, kv_out, q_out,
                 idx_s, idx_d, bkva, bkvb, bqa, bqb,
                 ska, skb, sqa, sqb):
        wid = lax.axis_index("s") * NC + lax.axis_index("c")
        base = wid * per_w
        pltpu.sync_copy(src_hbm.at[pl.ds(base, per_w)], idx_s)
        pltpu.sync_copy(dst_hbm.at[pl.ds(base, per_w)], idx_d)

        def fire(j, bkv, bq, skv, sq):
            o = j * CHG
            pltpu.async_copy(kv_hbm.at[idx_s.at[pl.ds(o, CHG)]], bkv, skv)
            pltpu.async_copy(q_hbm.at[idx_d.at[pl.ds(o, CHG)]], bq, sq)

        def finish(j, bkv, bq, skv, sq):
            o = j * CHG
            pltpu.make_async_copy(
                kv_hbm.at[idx_s.at[pl.ds(o, CHG)]], bkv, skv).wait()
            pltpu.make_async_copy(
                q_hbm.at[idx_d.at[pl.ds(o, CHG)]], bq, sq).wait()
            pltpu.sync_copy(bkv, kv_out.at[pl.ds(base + o, CHG)])
            pltpu.sync_copy(bq, q_out.at[pl.ds(base + o, CHG)])

        fire(0, bkva, bqa, ska, sqa)

        def body(i, carry):
            j = 2 * i
            fire(j + 1, bkvb, bqb, skb, sqb)
            finish(j, bkva, bqa, ska, sqa)
            fire(j + 2, bkva, bqa, ska, sqa)
            finish(j + 1, bkvb, bqb, skb, sqb)
            return carry

        lax.fori_loop(0, (full - 2) // 2, body, 0)
        fire(full - 1, bkvb, bqb, skb, sqb)
        finish(full - 2, bkva, bqa, ska, sqa)
        finish(full - 1, bkvb, bqb, skb, sqb)
        if tail:
            to = full * CHG
            pltpu.async_copy(kv_hbm.at[idx_s.at[pl.ds(to, tail)]],
                             bkva.at[pl.ds(0, tail)], ska)
            pltpu.async_copy(q_hbm.at[idx_d.at[pl.ds(to, tail)]],
                             bqa.at[pl.ds(0, tail)], sqa)
            pltpu.make_async_copy(kv_hbm.at[idx_s.at[pl.ds(to, tail)]],
                                  bkva.at[pl.ds(0, tail)], ska).wait()
            pltpu.make_async_copy(q_hbm.at[idx_d.at[pl.ds(to, tail)]],
                                  bqa.at[pl.ds(0, tail)], sqa).wait()
            pltpu.sync_copy(bkva.at[pl.ds(0, tail)],
                            kv_out.at[pl.ds(base + to, tail)])
            pltpu.sync_copy(bqa.at[pl.ds(0, tail)],
                            q_out.at[pl.ds(base + to, tail)])

    return gather_k(KV, Q, src, dst)


def _sc_scatter(data, dst, N):
    NE, W = data.shape
    per_w = NE // NW
    CHG = 128
    full = per_w // CHG
    tail = per_w - full * CHG
    RC = 80
    RT = 640
    zeros_w = jnp.zeros((RC, W), jnp.float32)
    mesh = plsc.VectorSubcoreMesh(core_axis_name="c", subcore_axis_name="s")

    @functools.partial(
        pl.kernel,
        out_type=jax.ShapeDtypeStruct((NC, N, W), jnp.float32),
        mesh=mesh,
        scratch_types=[
            pltpu.VMEM((per_w,), jnp.int32),
            pltpu.VMEM((CHG, W), jnp.float32),
            pltpu.VMEM((CHG, W), jnp.float32),
            pltpu.VMEM_SHARED((N, W), jnp.float32),
            pltpu.SemaphoreType.DMA,
            pltpu.SemaphoreType.DMA,
        ],
    )
    def scatter_k(d_hbm, dst_hbm, z_hbm, out, idx, eba, ebb,
                  acc, sfa, sfb):
        cid = lax.axis_index("c")
        sid = lax.axis_index("s")
        wid = sid * NC + cid
        row0 = sid * RT
        n_chunks = jnp.minimum(N - row0, RT) // RC
        base = wid * per_w

        ib = eba.at[pl.ds(0, RC)]
        pltpu.sync_copy(z_hbm, ib)

        def initj(j, carry):
            pltpu.sync_copy(ib, acc.at[pl.ds(row0 + j * RC, RC)])
            return carry

        lax.fori_loop(0, n_chunks, initj, 0)
        pltpu.sync_copy(dst_hbm.at[pl.ds(base, per_w)], idx)
        plsc.subcore_barrier()

        def fetch(j, eb, sf):
            pltpu.async_copy(d_hbm.at[pl.ds(base + j * CHG, CHG)], eb, sf)

        def wait_fetch(j, eb, sf):
            pltpu.make_async_copy(
                d_hbm.at[pl.ds(base + j * CHG, CHG)], eb, sf).wait()

        def add(j, eb):
            pltpu.sync_copy(eb, acc.at[idx.at[pl.ds(j * CHG, CHG)]],
                            add=True)

        fetch(0, eba, sfa)

        def body(i, carry):
            j = 2 * i
            fetch(j + 1, ebb, sfb)
            wait_fetch(j, eba, sfa)
            add(j, eba)
            fetch(j + 2, eba, sfa)
            wait_fetch(j + 1, ebb, sfb)
            add(j + 1, ebb)
            return carry

        lax.fori_loop(0, (full - 2) // 2, body, 0)
        fetch(full - 1, ebb, sfb)
        wait_fetch(full - 2, eba, sfa)
        add(full - 2, eba)
        wait_fetch(full - 1, ebb, sfb)
        add(full - 1, ebb)
        if tail:
            to = full * CHG
            pltpu.async_copy(d_hbm.at[pl.ds(base + to, tail)],
                             eba.at[pl.ds(0, tail)], sfa)
            pltpu.make_async_copy(d_hbm.at[pl.ds(base + to, tail)],
                                  eba.at[pl.ds(0, tail)], sfa).wait()
            pltpu.sync_copy(eba.at[pl.ds(0, tail)],
                            acc.at[idx.at[pl.ds(to, tail)]], add=True)
        plsc.subcore_barrier()

        def outj(j, carry):
            r = row0 + j * RC
            pltpu.sync_copy(acc.at[pl.ds(r, RC)], ib)
            pltpu.sync_copy(ib, out.at[cid, pl.ds(r, RC)])
            return carry

        lax.fori_loop(0, n_chunks, outj, 0)

    return scatter_k(data, dst, zeros_w)


def _dot(a, b):
    return jnp.dot(a, b, preferred_element_type=jnp.float32)


def _pack_kv(k, v):
    ku = lax.bitcast_convert_type(k.astype(jnp.bfloat16),
                                  jnp.uint16).astype(jnp.uint32)
    vu = lax.bitcast_convert_type(v.astype(jnp.bfloat16),
                                  jnp.uint16).astype(jnp.uint32)
    return lax.bitcast_convert_type((ku << 16) | vu, jnp.float32)


def _unpack_kv(p):
    u = lax.bitcast_convert_type(p, jnp.uint32)
    kb = lax.bitcast_convert_type((u >> 16).astype(jnp.uint16), jnp.bfloat16)
    vb = lax.bitcast_convert_type(u.astype(jnp.uint16), jnp.bfloat16)
    return kb.astype(jnp.float32), vb.astype(jnp.float32)


def _full(shape=None):
    return pl.BlockSpec(memory_space=pltpu.ANY) if shape is None else \
        pl.BlockSpec(shape, lambda i: (0,) * len(shape))


def _rows(shape):
    return pl.BlockSpec(shape, lambda i: (i,) + (0,) * (len(shape) - 1))


def _tc_embed_qkv(h, embW, embb, qW, qb, kW, kb, vW, vb):
    N = h.shape[0]

    def body(h_ref, ew_ref, eb_ref, qw_ref, qb_ref, kw_ref, kb_ref,
             vw_ref, vb_ref, hh_ref, q_ref, kv_ref):
        hh = _dot(h_ref[...], ew_ref[...]) + eb_ref[...]
        hh_ref[...] = hh
        q_ref[...] = _dot(hh, qw_ref[...]) + qb_ref[...]
        kv_ref[...] = _pack_kv(_dot(hh, kw_ref[...]) + kb_ref[...],
                               _dot(hh, vw_ref[...]) + vb_ref[...])

    return pl.pallas_call(
        body,
        out_shape=[jax.ShapeDtypeStruct((N, D), jnp.float32)] * 3,
    )(h, embW, embb, qW, qb, kW, kb, vW, vb)


def _tc_edge_full(kvsrc, qdst, e, embW, embb, eW, eb, oeW, oeb,
                  ones_hb, ones_hb_t):
    NE = kvsrc.shape[0]
    F = e.shape[1]
    grid = NE // RB

    def body(kv_ref, qd_ref, e_ref, ew_ref, ebias_ref, pw_ref, pb_ref,
             ow_ref, ob_ref, hb_ref, hbt_ref,
             wva_ref, wvb_ref, epre_ref, st_ref):
        ksrc, vsrc = _unpack_kv(kv_ref[...])
        ee = _dot(e_ref[...], ew_ref[...]) + ebias_ref[...]
        ep = _dot(ee, pw_ref[...]) + pb_ref[...]
        score = ksrc * qd_ref[...] * ep * 0.25
        ssum = _dot(score, hbt_ref[...])
        attn = jnp.exp(jnp.clip(ssum, -5.0, 5.0))
        attnb = _dot(attn, hb_ref[...])
        wv = vsrc * attnb
        pad = jnp.zeros((RB, WA - HD - H), jnp.float32)
        wva_ref[...] = jnp.concatenate([wv[:, :HD], attn, pad], axis=1)
        wvb_ref[...] = jnp.concatenate([wv[:, HD:], attn, pad], axis=1)
        epre = ee + _dot(score, ow_ref[...]) + ob_ref[...]
        epre_ref[...] = epre

        @pl.when(pl.program_id(0) == 0)
        def _():
            st_ref[...] = jnp.zeros_like(st_ref)

        s = jnp.sum(epre, axis=0)
        ss = jnp.sum(epre * epre, axis=0)
        st_ref[...] += jnp.concatenate(
            [s[None], ss[None], jnp.zeros((6, D), jnp.float32)], axis=0)

    return pl.pallas_call(
        body,
        grid=(grid,),
        in_specs=[_rows((RB, D)), _rows((RB, D)), _rows((RB, F)),
                  _full((F, D)), _full((1, D)), _full((D, D)), _full((1, D)),
                  _full((D, D)), _full((1, D)), _full((H, D)), _full((D, H))],
        out_specs=[_rows((RB, WA)), _rows((RB, WA)), _rows((RB, D)),
                   _full((8, D))],
        out_shape=[jax.ShapeDtypeStruct((NE, WA), jnp.float32),
                   jax.ShapeDtypeStruct((NE, WA), jnp.float32),
                   jax.ShapeDtypeStruct((NE, D), jnp.float32),
                   jax.ShapeDtypeStruct((8, D), jnp.float32)],
    )(kvsrc, qdst, e, embW, embb, eW, eb, oeW, oeb, ones_hb, ones_hb_t)


def _tc_edge_lite(kvsrc, qdst, y, stats, g2, b2, eW, eb, n_rows,
                  ones_hb, ones_hb_t):
    NE = kvsrc.shape[0]
    grid = NE // RB
    inv_n = 1.0 / float(n_rows)

    def body(kv_ref, qd_ref, y_ref, st_ref, g_ref, b_ref, pw_ref, pb_ref,
             hb_ref, hbt_ref, wva_ref, wvb_ref):
        ksrc, vsrc = _unpack_kv(kv_ref[...])
        mean = st_ref[0:1, :] * inv_n
        var = st_ref[1:2, :] * inv_n - mean * mean
        inv = lax.rsqrt(var + 1e-5)
        x = (y_ref[...] - mean) * inv * g_ref[...] + b_ref[...]
        ep = _dot(x, pw_ref[...]) + pb_ref[...]
        score = ksrc * qd_ref[...] * ep * 0.25
        attn = jnp.exp(jnp.clip(_dot(score, hbt_ref[...]), -5.0, 5.0))
        attnb = _dot(attn, hb_ref[...])
        wv = vsrc * attnb
        pad = jnp.zeros((RB, WA - HD - H), jnp.float32)
        wva_ref[...] = jnp.concatenate([wv[:, :HD], attn, pad], axis=1)
        wvb_ref[...] = jnp.concatenate([wv[:, HD:], attn, pad], axis=1)

    return pl.pallas_call(
        body,
        grid=(grid,),
        in_specs=[_rows((RB, D))] * 3
        + [_full((8, D)), _full((1, D)), _full((1, D)),
           _full((D, D)), _full((1, D)), _full((H, D)), _full((D, H))],
        out_specs=[_rows((RB, WA)), _rows((RB, WA))],
        out_shape=[jax.ShapeDtypeStruct((NE, WA), jnp.float32),
                   jax.ShapeDtypeStruct((NE, WA), jnp.float32)],
    )(kvsrc, qdst, y, stats, g2, b2, eW, eb, ones_hb, ones_hb_t)


def _tc_edge_ffn(epre, stats, g1, b1, w1, bb1, w2, bb2, n_rows):
    NE = epre.shape[0]
    grid = NE // RB
    inv_n = 1.0 / float(n_rows)

    def body(ep_ref, st_ref, g_ref, b_ref, w1_ref, b1_ref, w2_ref, b2_ref,
             y_ref, sy_ref):
        mean = st_ref[0:1, :] * inv_n
        var = st_ref[1:2, :] * inv_n - mean * mean
        inv = lax.rsqrt(var + 1e-5)
        x = (ep_ref[...] - mean) * inv * g_ref[...] + b_ref[...]
        hmid = jnp.maximum(_dot(x, w1_ref[...]) + b1_ref[...], 0.0)
        y = x + _dot(hmid, w2_ref[...]) + b2_ref[...]
        y_ref[...] = y

        @pl.when(pl.program_id(0) == 0)
        def _():
            sy_ref[...] = jnp.zeros_like(sy_ref)

        s = jnp.sum(y, axis=0)
        ss = jnp.sum(y * y, axis=0)
        sy_ref[...] += jnp.concatenate(
            [s[None], ss[None], jnp.zeros((6, D), jnp.float32)], axis=0)

    return pl.pallas_call(
        body,
        grid=(grid,),
        in_specs=[_rows((RB, D)), _full((8, D)), _full((1, D)), _full((1, D)),
                  _full((D, 2 * D)), _full((1, 2 * D)),
                  _full((2 * D, D)), _full((1, D))],
        out_specs=[_rows((RB, D)), _full((8, D))],
        out_shape=[jax.ShapeDtypeStruct((NE, D), jnp.float32),
                   jax.ShapeDtypeStruct((8, D), jnp.float32)],
    )(epre, stats, g1, b1, w1, bb1, w2, bb2)


def _node_update(hpa, hpb, hh_in, lp, ones_hb):
    a = hpa[0] + hpa[1]
    b = hpb[0] + hpb[1]
    wv = jnp.concatenate([a[:, :HD], b[:, :HD]], axis=1)
    z = a[:, HD:HD + H]
    r = 1.0 / (z + 1e-6)
    h_att = wv * _dot(r, ones_hb)
    h_new = _dot(h_att, lp['oh_w']) + lp['oh_b'] + hh_in
    m = jnp.mean(h_new, axis=0, keepdims=True)
    v = jnp.mean(h_new * h_new, axis=0, keepdims=True) - m * m
    h_new = (h_new - m) * lax.rsqrt(v + 1e-5) * lp['g1'] + lp['b1']
    h2 = _dot(jnp.maximum(_dot(h_new, lp['f1w']) + lp['f1b'], 0.0),
              lp['f2w']) + lp['f2b']
    h_new = h_new + h2
    m = jnp.mean(h_new, axis=0, keepdims=True)
    v = jnp.mean(h_new * h_new, axis=0, keepdims=True) - m * m
    return (h_new - m) * lax.rsqrt(v + 1e-5) * lp['g2'] + lp['b2']


def _tc_node_mid(hpart, zpart, hh_in, lw, qW, qb, kW, kb, vW, vb, ones_hb):
    N = hh_in.shape[0]

    def body(hp_ref, zp_ref, hh_ref, ow_ref, ob_ref, g1_ref, b1_ref,
             f1w_ref, f1b_ref, f2w_ref, f2b_ref, g2_ref, b2_ref,
             qw_ref, qb_ref, kw_ref, kb_ref, vw_ref, vb_ref, hb_ref,
             hh_out, q_ref, kv_ref):
        lp = dict(oh_w=ow_ref[...], oh_b=ob_ref[...], g1=g1_ref[...],
                  b1=b1_ref[...], f1w=f1w_ref[...], f1b=f1b_ref[...],
                  f2w=f2w_ref[...], f2b=f2b_ref[...], g2=g2_ref[...],
                  b2=b2_ref[...])
        hh1 = _node_update(hp_ref[...], zp_ref[...], hh_ref[...], lp,
                           hb_ref[...])
        hh_out[...] = hh1
        q_ref[...] = _dot(hh1, qw_ref[...]) + qb_ref[...]
        kv_ref[...] = _pack_kv(_dot(hh1, kw_ref[...]) + kb_ref[...],
                               _dot(hh1, vw_ref[...]) + vb_ref[...])

    return pl.pallas_call(
        body,
        out_shape=[jax.ShapeDtypeStruct((N, D), jnp.float32)] * 3,
    )(hpart, zpart, hh_in, lw['oh_w'], lw['oh_b'], lw['g1'], lw['b1'],
      lw['f1w'], lw['f1b'], lw['f2w'], lw['f2b'], lw['g2'], lw['b2'],
      qW, qb, kW, kb, vW, vb, ones_hb)


def _tc_node_head(hpart, zpart, hh_in, lw, w_top, w_bot, mlp1b, mlp2w,
                  mlp2b, vid, ones_hb, num_nodes):
    N = hh_in.shape[0]
    bs = vid.shape[0]

    def body(hp_ref, zp_ref, hh_ref, ow_ref, ob_ref, g1_ref, b1_ref,
             f1w_ref, f1b_ref, f2w_ref, f2b_ref, g2_ref, b2_ref,
             wt_ref, wb_ref, m1b_ref, m2w_ref, m2b_ref, hb_ref, vid_ref,
             out_ref, hh2_ref):
        lp = dict(oh_w=ow_ref[...], oh_b=ob_ref[...], g1=g1_ref[...],
                  b1=b1_ref[...], f1w=f1w_ref[...], f1b=f1b_ref[...],
                  f2w=f2w_ref[...], f2b=f2b_ref[...], g2=g2_ref[...],
                  b2=b2_ref[...])
        hh2 = _node_update(hp_ref[...], zp_ref[...], hh_ref[...], lp,
                           hb_ref[...])
        hh2_ref[...] = hh2
        for b in range(bs):
            row = hh2_ref[pl.ds(vid_ref[b], 1), :]
            cb = _dot(row, wt_ref[...]) + m1b_ref[...]
            blk = hh2_ref[pl.ds(b * num_nodes, num_nodes), :]
            t = jnp.maximum(_dot(blk, wb_ref[...]) + cb, 0.0)
            out_ref[pl.ds(b * num_nodes, num_nodes), :] = (
                _dot(t, m2w_ref[...]) + m2b_ref[...])

    return pl.pallas_call(
        body,
        in_specs=[pl.BlockSpec(memory_space=pltpu.VMEM)] * 19
        + [pl.BlockSpec(memory_space=pltpu.SMEM)],
        out_specs=pl.BlockSpec(memory_space=pltpu.VMEM),
        out_shape=jax.ShapeDtypeStruct((N, 1), jnp.float32),
        scratch_shapes=[pltpu.VMEM((N, D), jnp.float32)],
    )(hpart, zpart, hh_in, lw['oh_w'], lw['oh_b'], lw['g1'], lw['b1'],
      lw['f1w'], lw['f1b'], lw['f2w'], lw['f2b'], lw['g2'], lw['b2'],
      w_top, w_bot, mlp1b, mlp2w, mlp2b, ones_hb, vid)


def _row(x):
    return x.reshape(1, -1)


def _layer_w(lp):
    return dict(oh_w=lp['Oh_W'], oh_b=_row(lp['Oh_b']),
                g1=_row(lp['bn1h_g']), b1=_row(lp['bn1h_b']),
                f1w=lp['ffh1_W'], f1b=_row(lp['ffh1_b']),
                f2w=lp['ffh2_W'], f2b=_row(lp['ffh2_b']),
                g2=_row(lp['bn2h_g']), b2=_row(lp['bn2h_b']))


def kernel(h, e, params, edge_index, vehicle_node_id, batch_size):
    src = edge_index[0]
    dst = edge_index[1]
    N = h.shape[0]
    NE = src.shape[0]
    bs = vehicle_node_id.shape[0]
    num_nodes = N // bs
    L1, L2 = params['layers']

    ones_hb = jnp.repeat(jnp.eye(H, dtype=jnp.float32), DH, axis=1)
    ones_hb_t = ones_hb.T

    hh0, q1, kv1 = _tc_embed_qkv(
        h, params['emb_h_W'], _row(params['emb_h_b']),
        L1['Q_W'], _row(L1['Q_b']), L1['K_W'], _row(L1['K_b']),
        L1['V_W'], _row(L1['V_b']))

    kvsrc1, qdst1 = _sc_gather(kv1, q1, src, dst)
    wv1a, wv1b, epre1, st1 = _tc_edge_full(
        kvsrc1, qdst1, e, params['emb_e_W'], _row(params['emb_e_b']),
        L1['E_W'], _row(L1['E_b']), L1['Oe_W'], _row(L1['Oe_b']),
        ones_hb, ones_hb_t)
    hpa1 = _sc_scatter(wv1a, dst, N)
    hpb1 = _sc_scatter(wv1b, dst, N)
    hh1, q2, kv2 = _tc_node_mid(
        hpa1, hpb1, hh0, _layer_w(L1),
        L2['Q_W'], _row(L2['Q_b']), L2['K_W'], _row(L2['K_b']),
        L2['V_W'], _row(L2['V_b']), ones_hb)
    y1, sty1 = _tc_edge_ffn(
        epre1, st1, _row(L1['bn1e_g']), _row(L1['bn1e_b']),
        L1['ffe1_W'], _row(L1['ffe1_b']), L1['ffe2_W'], _row(L1['ffe2_b']),
        NE)

    kvsrc2, qdst2 = _sc_gather(kv2, q2, src, dst)
    wv2a, wv2b = _tc_edge_lite(
        kvsrc2, qdst2, y1, sty1, _row(L1['bn2e_g']), _row(L1['bn2e_b']),
        L2['E_W'], _row(L2['E_b']), NE, ones_hb, ones_hb_t)
    hpa2 = _sc_scatter(wv2a, dst, N)
    hpb2 = _sc_scatter(wv2b, dst, N)

    vid = (vehicle_node_id.astype(jnp.int32)
           + jnp.arange(bs, dtype=jnp.int32) * num_nodes
           + (jnp.asarray(batch_size, jnp.int32) - jnp.int32(bs)))
    policy = _tc_node_head(
        hpa2, hpb2, hh1, _layer_w(L2),
        params['mlp1_W'][:D], params['mlp1_W'][D:], _row(params['mlp1_b']),
        params['mlp2_W'], _row(params['mlp2_b']), vid, ones_hb, num_nodes)
    return policy[:, 0].reshape(bs, num_nodes)


# speedup vs baseline: 1.2613x; 1.0001x over previous
import functools

import jax
import jax.numpy as jnp
from jax import lax
from jax.experimental import pallas as pl
from jax.experimental.pallas import tpu as pltpu
from jax.experimental.pallas import tpu_sc as plsc

D = 128
H = 8
DH = 16
HD = 64
WA = 128
NC = 2
NS = 16
NW = NC * NS
CH = 80
RB = 2000


def _sc_gather(KV, Q, src, dst):
    N, DKV = KV.shape
    NE = src.shape[0]
    per_w = NE // NW
    CHG = 128
    full = per_w // CHG
    tail = per_w - full * CHG
    mesh = plsc.VectorSubcoreMesh(core_axis_name="c", subcore_axis_name="s")

    @functools.partial(
        pl.kernel,
        out_type=[jax.ShapeDtypeStruct((NE, DKV), jnp.float32),
                  jax.ShapeDtypeStruct((NE, D), jnp.float32)],
        mesh=mesh,
        scratch_types=[
            pltpu.VMEM((per_w,), jnp.int32),
            pltpu.VMEM((per_w,), jnp.int32),
            pltpu.VMEM((CHG, DKV), jnp.float32),
            pltpu.VMEM((CHG, DKV), jnp.float32),
            pltpu.VMEM((CHG, DKV), jnp.float32),
            pltpu.VMEM((CHG, D), jnp.float32),
            pltpu.VMEM((CHG, D), jnp.float32),
            pltpu.VMEM((CHG, D), jnp.float32),
            pltpu.SemaphoreType.DMA,
            pltpu.SemaphoreType.DMA,
            pltpu.SemaphoreType.DMA,
            pltpu.SemaphoreType.DMA,
            pltpu.SemaphoreType.DMA,
            pltpu.SemaphoreType.DMA,
        ],
    )
    def gather_k(kv_hbm, q_hbm, src_hbm, dst_hbm, kv_out, q_out,
                 idx_s, idx_d, bkva, bkvb, bkvc, bqa, bqb, bqc,
                 ska, skb, skc, sqa, sqb, sqc):
        wid = lax.axis_index("s") * NC + lax.axis_index("c")
        base = wid * per_w
        pltpu.sync_copy(src_hbm.at[pl.ds(base, per_w)], idx_s)
        pltpu.sync_copy(dst_hbm.at[pl.ds(base, per_w)], idx_d)

        def fire(j, bkv, bq, skv, sq):
            o = j * CHG
            pltpu.async_copy(kv_hbm.at[idx_s.at[pl.ds(o, CHG)]], bkv, skv)
            pltpu.async_copy(q_hbm.at[idx_d.at[pl.ds(o, CHG)]], bq, sq)

        def finish(j, bkv, bq, skv, sq):
            o = j * CHG
            pltpu.make_async_copy(
                kv_hbm.at[idx_s.at[pl.ds(o, CHG)]], bkv, skv).wait()
            pltpu.make_async_copy(
                q_hbm.at[idx_d.at[pl.ds(o, CHG)]], bq, sq).wait()
            pltpu.sync_copy(bkv, kv_out.at[pl.ds(base + o, CHG)])
            pltpu.sync_copy(bq, q_out.at[pl.ds(base + o, CHG)])

        A = (bkva, bqa, ska, sqa)
        B = (bkvb, bqb, skb, sqb)
        C = (bkvc, bqc, skc, sqc)
        fire(0, *A)
        fire(1, *B)

        def body(i, carry):
            j = 3 * i
            fire(j + 2, *C)
            finish(j, *A)
            fire(j + 3, *A)
            finish(j + 1, *B)
            fire(j + 4, *B)
            finish(j + 2, *C)
            return carry

        lax.fori_loop(0, (full - 3) // 3, body, 0)
        fire(full - 1, *C)
        finish(full - 3, *A)
        finish(full - 2, *B)
        finish(full - 1, *C)
        if tail:
            to = full * CHG
            pltpu.async_copy(kv_hbm.at[idx_s.at[pl.ds(to, tail)]],
                             bkva.at[pl.ds(0, tail)], ska)
            pltpu.async_copy(q_hbm.at[idx_d.at[pl.ds(to, tail)]],
                             bqa.at[pl.ds(0, tail)], sqa)
            pltpu.make_async_copy(kv_hbm.at[idx_s.at[pl.ds(to, tail)]],
                                  bkva.at[pl.ds(0, tail)], ska).wait()
            pltpu.make_async_copy(q_hbm.at[idx_d.at[pl.ds(to, tail)]],
                                  bqa.at[pl.ds(0, tail)], sqa).wait()
            pltpu.sync_copy(bkva.at[pl.ds(0, tail)],
                            kv_out.at[pl.ds(base + to, tail)])
            pltpu.sync_copy(bqa.at[pl.ds(0, tail)],
                            q_out.at[pl.ds(base + to, tail)])

    return gather_k(KV, Q, src, dst)


def _sc_scatter(data, dst, N):
    NE, W = data.shape
    per_w = NE // NW
    CHG = 128
    full = per_w // CHG
    tail = per_w - full * CHG
    RC = 80
    RT = 640
    zeros_w = jnp.zeros((RC, W), jnp.float32)
    mesh = plsc.VectorSubcoreMesh(core_axis_name="c", subcore_axis_name="s")

    @functools.partial(
        pl.kernel,
        out_type=jax.ShapeDtypeStruct((NC, N, W), jnp.float32),
        mesh=mesh,
        scratch_types=[
            pltpu.VMEM((per_w,), jnp.int32),
            pltpu.VMEM((CHG, W), jnp.float32),
            pltpu.VMEM((CHG, W), jnp.float32),
            pltpu.VMEM_SHARED((N, W), jnp.float32),
            pltpu.SemaphoreType.DMA,
            pltpu.SemaphoreType.DMA,
        ],
    )
    def scatter_k(d_hbm, dst_hbm, z_hbm, out, idx, eba, ebb,
                  acc, sfa, sfb):
        cid = lax.axis_index("c")
        sid = lax.axis_index("s")
        wid = sid * NC + cid
        row0 = sid * RT
        n_chunks = jnp.minimum(N - row0, RT) // RC
        base = wid * per_w

        ib = eba.at[pl.ds(0, RC)]
        pltpu.sync_copy(z_hbm, ib)

        def initj(j, carry):
            pltpu.sync_copy(ib, acc.at[pl.ds(row0 + j * RC, RC)])
            return carry

        lax.fori_loop(0, n_chunks, initj, 0)
        pltpu.sync_copy(dst_hbm.at[pl.ds(base, per_w)], idx)
        plsc.subcore_barrier()

        def fetch(j, eb, sf):
            pltpu.async_copy(d_hbm.at[pl.ds(base + j * CHG, CHG)], eb, sf)

        def wait_fetch(j, eb, sf):
            pltpu.make_async_copy(
                d_hbm.at[pl.ds(base + j * CHG, CHG)], eb, sf).wait()

        def add(j, eb):
            pltpu.sync_copy(eb, acc.at[idx.at[pl.ds(j * CHG, CHG)]],
                            add=True)

        fetch(0, eba, sfa)

        def body(i, carry):
            j = 2 * i
            fetch(j + 1, ebb, sfb)
            wait_fetch(j, eba, sfa)
            add(j, eba)
            fetch(j + 2, eba, sfa)
            wait_fetch(j + 1, ebb, sfb)
            add(j + 1, ebb)
            return carry

        lax.fori_loop(0, (full - 2) // 2, body, 0)
        fetch(full - 1, ebb, sfb)
        wait_fetch(full - 2, eba, sfa)
        add(full - 2, eba)
        wait_fetch(full - 1, ebb, sfb)
        add(full - 1, ebb)
        if tail:
            to = full * CHG
            pltpu.async_copy(d_hbm.at[pl.ds(base + to, tail)],
                             eba.at[pl.ds(0, tail)], sfa)
            pltpu.make_async_copy(d_hbm.at[pl.ds(base + to, tail)],
                                  eba.at[pl.ds(0, tail)], sfa).wait()
            pltpu.sync_copy(eba.at[pl.ds(0, tail)],
                            acc.at[idx.at[pl.ds(to, tail)]], add=True)
        plsc.subcore_barrier()

        def outj(j, carry):
            r = row0 + j * RC
            pltpu.sync_copy(acc.at[pl.ds(r, RC)], ib)
            pltpu.sync_copy(ib, out.at[cid, pl.ds(r, RC)])
            return carry

        lax.fori_loop(0, n_chunks, outj, 0)

    return scatter_k(data, dst, zeros_w)


def _dot(a, b):
    return jnp.dot(a, b, preferred_element_type=jnp.float32)


def _pack_kv(k, v):
    ku = lax.bitcast_convert_type(k.astype(jnp.bfloat16),
                                  jnp.uint16).astype(jnp.uint32)
    vu = lax.bitcast_convert_type(v.astype(jnp.bfloat16),
                                  jnp.uint16).astype(jnp.uint32)
    return lax.bitcast_convert_type((ku << 16) | vu, jnp.float32)


def _unpack_kv(p):
    u = lax.bitcast_convert_type(p, jnp.uint32)
    kb = lax.bitcast_convert_type((u >> 16).astype(jnp.uint16), jnp.bfloat16)
    vb = lax.bitcast_convert_type(u.astype(jnp.uint16), jnp.bfloat16)
    return kb.astype(jnp.float32), vb.astype(jnp.float32)


def _full(shape=None):
    return pl.BlockSpec(memory_space=pltpu.ANY) if shape is None else \
        pl.BlockSpec(shape, lambda i: (0,) * len(shape))


def _rows(shape):
    return pl.BlockSpec(shape, lambda i: (i,) + (0,) * (len(shape) - 1))


def _tc_embed_qkv(h, embW, embb, qW, qb, kW, kb, vW, vb):
    N = h.shape[0]

    def body(h_ref, ew_ref, eb_ref, qw_ref, qb_ref, kw_ref, kb_ref,
             vw_ref, vb_ref, hh_ref, q_ref, kv_ref):
        hh = _dot(h_ref[...], ew_ref[...]) + eb_ref[...]
        hh_ref[...] = hh
        q_ref[...] = _dot(hh, qw_ref[...]) + qb_ref[...]
        kv_ref[...] = _pack_kv(_dot(hh, kw_ref[...]) + kb_ref[...],
                               _dot(hh, vw_ref[...]) + vb_ref[...])

    return pl.pallas_call(
        body,
        out_shape=[jax.ShapeDtypeStruct((N, D), jnp.float32)] * 3,
    )(h, embW, embb, qW, qb, kW, kb, vW, vb)


def _tc_edge_full(kvsrc, qdst, e, embW, embb, eW, eb, oeW, oeb,
                  ones_hb, ones_hb_t):
    NE = kvsrc.shape[0]
    F = e.shape[1]
    grid = NE // RB

    def body(kv_ref, qd_ref, e_ref, ew_ref, ebias_ref, pw_ref, pb_ref,
             ow_ref, ob_ref, hb_ref, hbt_ref,
             wva_ref, wvb_ref, epre_ref, st_ref):
        ksrc, vsrc = _unpack_kv(kv_ref[...])
        ee = _dot(e_ref[...], ew_ref[...]) + ebias_ref[...]
        ep = _dot(ee, pw_ref[...]) + pb_ref[...]
        score = ksrc * qd_ref[...] * ep * 0.25
        ssum = _dot(score, hbt_ref[...])
        attn = jnp.exp(jnp.clip(ssum, -5.0, 5.0))
        attnb = _dot(attn, hb_ref[...])
        wv = vsrc * attnb
        pad = jnp.zeros((RB, WA - HD - H), jnp.float32)
        wva_ref[...] = jnp.concatenate([wv[:, :HD], attn, pad], axis=1)
        wvb_ref[...] = jnp.concatenate([wv[:, HD:], attn, pad], axis=1)
        epre = ee + _dot(score, ow_ref[...]) + ob_ref[...]
        epre_ref[...] = epre

        @pl.when(pl.program_id(0) == 0)
        def _():
            st_ref[...] = jnp.zeros_like(st_ref)

        s = jnp.sum(epre, axis=0)
        ss = jnp.sum(epre * epre, axis=0)
        st_ref[...] += jnp.concatenate(
            [s[None], ss[None], jnp.zeros((6, D), jnp.float32)], axis=0)

    return pl.pallas_call(
        body,
        grid=(grid,),
        in_specs=[_rows((RB, D)), _rows((RB, D)), _rows((RB, F)),
                  _full((F, D)), _full((1, D)), _full((D, D)), _full((1, D)),
                  _full((D, D)), _full((1, D)), _full((H, D)), _full((D, H))],
        out_specs=[_rows((RB, WA)), _rows((RB, WA)), _rows((RB, D)),
                   _full((8, D))],
        out_shape=[jax.ShapeDtypeStruct((NE, WA), jnp.float32),
                   jax.ShapeDtypeStruct((NE, WA), jnp.float32),
                   jax.ShapeDtypeStruct((NE, D), jnp.float32),
                   jax.ShapeDtypeStruct((8, D), jnp.float32)],
    )(kvsrc, qdst, e, embW, embb, eW, eb, oeW, oeb, ones_hb, ones_hb_t)


def _tc_edge_lite(kvsrc, qdst, y, stats, g2, b2, eW, eb, n_rows,
                  ones_hb, ones_hb_t):
    NE = kvsrc.shape[0]
    grid = NE // RB
    inv_n = 1.0 / float(n_rows)

    def body(kv_ref, qd_ref, y_ref, st_ref, g_ref, b_ref, pw_ref, pb_ref,
             hb_ref, hbt_ref, wva_ref, wvb_ref):
        ksrc, vsrc = _unpack_kv(kv_ref[...])
        mean = st_ref[0:1, :] * inv_n
        var = st_ref[1:2, :] * inv_n - mean * mean
        inv = lax.rsqrt(var + 1e-5)
        x = (y_ref[...] - mean) * inv * g_ref[...] + b_ref[...]
        ep = _dot(x, pw_ref[...]) + pb_ref[...]
        score = ksrc * qd_ref[...] * ep * 0.25
        attn = jnp.exp(jnp.clip(_dot(score, hbt_ref[...]), -5.0, 5.0))
        attnb = _dot(attn, hb_ref[...])
        wv = vsrc * attnb
        pad = jnp.zeros((RB, WA - HD - H), jnp.float32)
        wva_ref[...] = jnp.concatenate([wv[:, :HD], attn, pad], axis=1)
        wvb_ref[...] = jnp.concatenate([wv[:, HD:], attn, pad], axis=1)

    return pl.pallas_call(
        body,
        grid=(grid,),
        in_specs=[_rows((RB, D))] * 3
        + [_full((8, D)), _full((1, D)), _full((1, D)),
           _full((D, D)), _full((1, D)), _full((H, D)), _full((D, H))],
        out_specs=[_rows((RB, WA)), _rows((RB, WA))],
        out_shape=[jax.ShapeDtypeStruct((NE, WA), jnp.float32),
                   jax.ShapeDtypeStruct((NE, WA), jnp.float32)],
    )(kvsrc, qdst, y, stats, g2, b2, eW, eb, ones_hb, ones_hb_t)


def _tc_edge_ffn(epre, stats, g1, b1, w1, bb1, w2, bb2, n_rows):
    NE = epre.shape[0]
    grid = NE // RB
    inv_n = 1.0 / float(n_rows)

    def body(ep_ref, st_ref, g_ref, b_ref, w1_ref, b1_ref, w2_ref, b2_ref,
             y_ref, sy_ref):
        mean = st_ref[0:1, :] * inv_n
        var = st_ref[1:2, :] * inv_n - mean * mean
        inv = lax.rsqrt(var + 1e-5)
        x = (ep_ref[...] - mean) * inv * g_ref[...] + b_ref[...]
        hmid = jnp.maximum(_dot(x, w1_ref[...]) + b1_ref[...], 0.0)
        y = x + _dot(hmid, w2_ref[...]) + b2_ref[...]
        y_ref[...] = y

        @pl.when(pl.program_id(0) == 0)
        def _():
            sy_ref[...] = jnp.zeros_like(sy_ref)

        s = jnp.sum(y, axis=0)
        ss = jnp.sum(y * y, axis=0)
        sy_ref[...] += jnp.concatenate(
            [s[None], ss[None], jnp.zeros((6, D), jnp.float32)], axis=0)

    return pl.pallas_call(
        body,
        grid=(grid,),
        in_specs=[_rows((RB, D)), _full((8, D)), _full((1, D)), _full((1, D)),
                  _full((D, 2 * D)), _full((1, 2 * D)),
                  _full((2 * D, D)), _full((1, D))],
        out_specs=[_rows((RB, D)), _full((8, D))],
        out_shape=[jax.ShapeDtypeStruct((NE, D), jnp.float32),
                   jax.ShapeDtypeStruct((8, D), jnp.float32)],
    )(epre, stats, g1, b1, w1, bb1, w2, bb2)


def _node_update(hpa, hpb, hh_in, lp, ones_hb):
    a = hpa[0] + hpa[1]
    b = hpb[0] + hpb[1]
    wv = jnp.concatenate([a[:, :HD], b[:, :HD]], axis=1)
    z = a[:, HD:HD + H]
    r = 1.0 / (z + 1e-6)
    h_att = wv * _dot(r, ones_hb)
    h_new = _dot(h_att, lp['oh_w']) + lp['oh_b'] + hh_in
    m = jnp.mean(h_new, axis=0, keepdims=True)
    v = jnp.mean(h_new * h_new, axis=0, keepdims=True) - m * m
    h_new = (h_new - m) * lax.rsqrt(v + 1e-5) * lp['g1'] + lp['b1']
    h2 = _dot(jnp.maximum(_dot(h_new, lp['f1w']) + lp['f1b'], 0.0),
              lp['f2w']) + lp['f2b']
    h_new = h_new + h2
    m = jnp.mean(h_new, axis=0, keepdims=True)
    v = jnp.mean(h_new * h_new, axis=0, keepdims=True) - m * m
    return (h_new - m) * lax.rsqrt(v + 1e-5) * lp['g2'] + lp['b2']


def _tc_node_mid(hpart, zpart, hh_in, lw, qW, qb, kW, kb, vW, vb, ones_hb):
    N = hh_in.shape[0]

    def body(hp_ref, zp_ref, hh_ref, ow_ref, ob_ref, g1_ref, b1_ref,
             f1w_ref, f1b_ref, f2w_ref, f2b_ref, g2_ref, b2_ref,
             qw_ref, qb_ref, kw_ref, kb_ref, vw_ref, vb_ref, hb_ref,
             hh_out, q_ref, kv_ref):
        lp = dict(oh_w=ow_ref[...], oh_b=ob_ref[...], g1=g1_ref[...],
                  b1=b1_ref[...], f1w=f1w_ref[...], f1b=f1b_ref[...],
                  f2w=f2w_ref[...], f2b=f2b_ref[...], g2=g2_ref[...],
                  b2=b2_ref[...])
        hh1 = _node_update(hp_ref[...], zp_ref[...], hh_ref[...], lp,
                           hb_ref[...])
        hh_out[...] = hh1
        q_ref[...] = _dot(hh1, qw_ref[...]) + qb_ref[...]
        kv_ref[...] = _pack_kv(_dot(hh1, kw_ref[...]) + kb_ref[...],
                               _dot(hh1, vw_ref[...]) + vb_ref[...])

    return pl.pallas_call(
        body,
        out_shape=[jax.ShapeDtypeStruct((N, D), jnp.float32)] * 3,
    )(hpart, zpart, hh_in, lw['oh_w'], lw['oh_b'], lw['g1'], lw['b1'],
      lw['f1w'], lw['f1b'], lw['f2w'], lw['f2b'], lw['g2'], lw['b2'],
      qW, qb, kW, kb, vW, vb, ones_hb)


def _tc_node_head(hpart, zpart, hh_in, lw, w_top, w_bot, mlp1b, mlp2w,
                  mlp2b, vid, ones_hb, num_nodes):
    N = hh_in.shape[0]
    bs = vid.shape[0]

    def body(hp_ref, zp_ref, hh_ref, ow_ref, ob_ref, g1_ref, b1_ref,
             f1w_ref, f1b_ref, f2w_ref, f2b_ref, g2_ref, b2_ref,
             wt_ref, wb_ref, m1b_ref, m2w_ref, m2b_ref, hb_ref, vid_ref,
             out_ref, hh2_ref):
        lp = dict(oh_w=ow_ref[...], oh_b=ob_ref[...], g1=g1_ref[...],
                  b1=b1_ref[...], f1w=f1w_ref[...], f1b=f1b_ref[...],
                  f2w=f2w_ref[...], f2b=f2b_ref[...], g2=g2_ref[...],
                  b2=b2_ref[...])
        hh2 = _node_update(hp_ref[...], zp_ref[...], hh_ref[...], lp,
                           hb_ref[...])
        hh2_ref[...] = hh2
        for b in range(bs):
            row = hh2_ref[pl.ds(vid_ref[b], 1), :]
            cb = _dot(row, wt_ref[...]) + m1b_ref[...]
            blk = hh2_ref[pl.ds(b * num_nodes, num_nodes), :]
            t = jnp.maximum(_dot(blk, wb_ref[...]) + cb, 0.0)
            out_ref[pl.ds(b * num_nodes, num_nodes), :] = (
                _dot(t, m2w_ref[...]) + m2b_ref[...])

    return pl.pallas_call(
        body,
        in_specs=[pl.BlockSpec(memory_space=pltpu.VMEM)] * 19
        + [pl.BlockSpec(memory_space=pltpu.SMEM)],
        out_specs=pl.BlockSpec(memory_space=pltpu.VMEM),
        out_shape=jax.ShapeDtypeStruct((N, 1), jnp.float32),
        scratch_shapes=[pltpu.VMEM((N, D), jnp.float32)],
    )(hpart, zpart, hh_in, lw['oh_w'], lw['oh_b'], lw['g1'], lw['b1'],
      lw['f1w'], lw['f1b'], lw['f2w'], lw['f2b'], lw['g2'], lw['b2'],
      w_top, w_bot, mlp1b, mlp2w, mlp2b, ones_hb, vid)


def _row(x):
    return x.reshape(1, -1)


def _layer_w(lp):
    return dict(oh_w=lp['Oh_W'], oh_b=_row(lp['Oh_b']),
                g1=_row(lp['bn1h_g']), b1=_row(lp['bn1h_b']),
                f1w=lp['ffh1_W'], f1b=_row(lp['ffh1_b']),
                f2w=lp['ffh2_W'], f2b=_row(lp['ffh2_b']),
                g2=_row(lp['bn2h_g']), b2=_row(lp['bn2h_b']))


def kernel(h, e, params, edge_index, vehicle_node_id, batch_size):
    src = edge_index[0]
    dst = edge_index[1]
    N = h.shape[0]
    NE = src.shape[0]
    bs = vehicle_node_id.shape[0]
    num_nodes = N // bs
    L1, L2 = params['layers']

    ones_hb = jnp.repeat(jnp.eye(H, dtype=jnp.float32), DH, axis=1)
    ones_hb_t = ones_hb.T

    hh0, q1, kv1 = _tc_embed_qkv(
        h, params['emb_h_W'], _row(params['emb_h_b']),
        L1['Q_W'], _row(L1['Q_b']), L1['K_W'], _row(L1['K_b']),
        L1['V_W'], _row(L1['V_b']))

    kvsrc1, qdst1 = _sc_gather(kv1, q1, src, dst)
    wv1a, wv1b, epre1, st1 = _tc_edge_full(
        kvsrc1, qdst1, e, params['emb_e_W'], _row(params['emb_e_b']),
        L1['E_W'], _row(L1['E_b']), L1['Oe_W'], _row(L1['Oe_b']),
        ones_hb, ones_hb_t)
    hpa1 = _sc_scatter(wv1a, dst, N)
    hpb1 = _sc_scatter(wv1b, dst, N)
    hh1, q2, kv2 = _tc_node_mid(
        hpa1, hpb1, hh0, _layer_w(L1),
        L2['Q_W'], _row(L2['Q_b']), L2['K_W'], _row(L2['K_b']),
        L2['V_W'], _row(L2['V_b']), ones_hb)
    y1, sty1 = _tc_edge_ffn(
        epre1, st1, _row(L1['bn1e_g']), _row(L1['bn1e_b']),
        L1['ffe1_W'], _row(L1['ffe1_b']), L1['ffe2_W'], _row(L1['ffe2_b']),
        NE)

    kvsrc2, qdst2 = _sc_gather(kv2, q2, src, dst)
    wv2a, wv2b = _tc_edge_lite(
        kvsrc2, qdst2, y1, sty1, _row(L1['bn2e_g']), _row(L1['bn2e_b']),
        L2['E_W'], _row(L2['E_b']), NE, ones_hb, ones_hb_t)
    hpa2 = _sc_scatter(wv2a, dst, N)
    hpb2 = _sc_scatter(wv2b, dst, N)

    vid = (vehicle_node_id.astype(jnp.int32)
           + jnp.arange(bs, dtype=jnp.int32) * num_nodes
           + (jnp.asarray(batch_size, jnp.int32) - jnp.int32(bs)))
    policy = _tc_node_head(
        hpa2, hpb2, hh1, _layer_w(L2),
        params['mlp1_W'][:D], params['mlp1_W'][D:], _row(params['mlp1_b']),
        params['mlp2_W'], _row(params['mlp2_b']), vid, ones_hb, num_nodes)
    return policy[:, 0].reshape(bs, num_nodes)


# speedup vs baseline: 1.3221x; 1.0482x over previous
import functools

import jax
import jax.numpy as jnp
from jax import lax
from jax.experimental import pallas as pl
from jax.experimental.pallas import tpu as pltpu
from jax.experimental.pallas import tpu_sc as plsc

D = 128
H = 8
DH = 16
HD = 64
WA = 128
NC = 2
NS = 16
NW = NC * NS
CH = 80
RB = 4000


def _sc_gather(KV, Q, src, dst):
    N, DKV = KV.shape
    NE = src.shape[0]
    per_w = NE // NW
    CHG = 128
    full = per_w // CHG
    tail = per_w - full * CHG
    mesh = plsc.VectorSubcoreMesh(core_axis_name="c", subcore_axis_name="s")

    @functools.partial(
        pl.kernel,
        out_type=[jax.ShapeDtypeStruct((NE, DKV), jnp.float32),
                  jax.ShapeDtypeStruct((NE, D), jnp.float32)],
        mesh=mesh,
        scratch_types=[
            pltpu.VMEM((per_w,), jnp.int32),
            pltpu.VMEM((per_w,), jnp.int32),
            pltpu.VMEM((CHG, DKV), jnp.float32),
            pltpu.VMEM((CHG, DKV), jnp.float32),
            pltpu.VMEM((CHG, DKV), jnp.float32),
            pltpu.VMEM((CHG, D), jnp.float32),
            pltpu.VMEM((CHG, D), jnp.float32),
            pltpu.VMEM((CHG, D), jnp.float32),
            pltpu.SemaphoreType.DMA,
            pltpu.SemaphoreType.DMA,
            pltpu.SemaphoreType.DMA,
            pltpu.SemaphoreType.DMA,
            pltpu.SemaphoreType.DMA,
            pltpu.SemaphoreType.DMA,
        ],
    )
    def gather_k(kv_hbm, q_hbm, src_hbm, dst_hbm, kv_out, q_out,
                 idx_s, idx_d, bkva, bkvb, bkvc, bqa, bqb, bqc,
                 ska, skb, skc, sqa, sqb, sqc):
        wid = lax.axis_index("s") * NC + lax.axis_index("c")
        base = wid * per_w
        pltpu.sync_copy(src_hbm.at[pl.ds(base, per_w)], idx_s)
        pltpu.sync_copy(dst_hbm.at[pl.ds(base, per_w)], idx_d)

        def fire(j, bkv, bq, skv, sq):
            o = j * CHG
            pltpu.async_copy(kv_hbm.at[idx_s.at[pl.ds(o, CHG)]], bkv, skv)
            pltpu.async_copy(q_hbm.at[idx_d.at[pl.ds(o, CHG)]], bq, sq)

        def finish(j, bkv, bq, skv, sq):
            o = j * CHG
            pltpu.make_async_copy(
                kv_hbm.at[idx_s.at[pl.ds(o, CHG)]], bkv, skv).wait()
            pltpu.make_async_copy(
                q_hbm.at[idx_d.at[pl.ds(o, CHG)]], bq, sq).wait()
            pltpu.sync_copy(bkv, kv_out.at[pl.ds(base + o, CHG)])
            pltpu.sync_copy(bq, q_out.at[pl.ds(base + o, CHG)])

        A = (bkva, bqa, ska, sqa)
        B = (bkvb, bqb, skb, sqb)
        C = (bkvc, bqc, skc, sqc)
        fire(0, *A)
        fire(1, *B)

        def body(i, carry):
            j = 3 * i
            fire(j + 2, *C)
            finish(j, *A)
            fire(j + 3, *A)
            finish(j + 1, *B)
            fire(j + 4, *B)
            finish(j + 2, *C)
            return carry

        lax.fori_loop(0, (full - 3) // 3, body, 0)
        fire(full - 1, *C)
        finish(full - 3, *A)
        finish(full - 2, *B)
        finish(full - 1, *C)
        if tail:
            to = full * CHG
            pltpu.async_copy(kv_hbm.at[idx_s.at[pl.ds(to, tail)]],
                             bkva.at[pl.ds(0, tail)], ska)
            pltpu.async_copy(q_hbm.at[idx_d.at[pl.ds(to, tail)]],
                             bqa.at[pl.ds(0, tail)], sqa)
            pltpu.make_async_copy(kv_hbm.at[idx_s.at[pl.ds(to, tail)]],
                                  bkva.at[pl.ds(0, tail)], ska).wait()
            pltpu.make_async_copy(q_hbm.at[idx_d.at[pl.ds(to, tail)]],
                                  bqa.at[pl.ds(0, tail)], sqa).wait()
            pltpu.sync_copy(bkva.at[pl.ds(0, tail)],
                            kv_out.at[pl.ds(base + to, tail)])
            pltpu.sync_copy(bqa.at[pl.ds(0, tail)],
                            q_out.at[pl.ds(base + to, tail)])

    return gather_k(KV, Q, src, dst)


def _sc_scatter(data, dst, N):
    NE, W = data.shape
    per_w = NE // NW
    CHG = 128
    full = per_w // CHG
    tail = per_w - full * CHG
    RC = 80
    RT = 640
    zeros_w = jnp.zeros((RC, W), jnp.float32)
    mesh = plsc.VectorSubcoreMesh(core_axis_name="c", subcore_axis_name="s")

    @functools.partial(
        pl.kernel,
        out_type=jax.ShapeDtypeStruct((NC, N, W), jnp.float32),
        mesh=mesh,
        scratch_types=[
            pltpu.VMEM((per_w,), jnp.int32),
            pltpu.VMEM((CHG, W), jnp.float32),
            pltpu.VMEM((CHG, W), jnp.float32),
            pltpu.VMEM_SHARED((N, W), jnp.float32),
            pltpu.SemaphoreType.DMA,
            pltpu.SemaphoreType.DMA,
        ],
    )
    def scatter_k(d_hbm, dst_hbm, z_hbm, out, idx, eba, ebb,
                  acc, sfa, sfb):
        cid = lax.axis_index("c")
        sid = lax.axis_index("s")
        wid = sid * NC + cid
        row0 = sid * RT
        n_chunks = jnp.minimum(N - row0, RT) // RC
        base = wid * per_w

        ib = eba.at[pl.ds(0, RC)]
        pltpu.sync_copy(z_hbm, ib)

        def initj(j, carry):
            pltpu.sync_copy(ib, acc.at[pl.ds(row0 + j * RC, RC)])
            return carry

        lax.fori_loop(0, n_chunks, initj, 0)
        pltpu.sync_copy(dst_hbm.at[pl.ds(base, per_w)], idx)
        plsc.subcore_barrier()

        def fetch(j, eb, sf):
            pltpu.async_copy(d_hbm.at[pl.ds(base + j * CHG, CHG)], eb, sf)

        def wait_fetch(j, eb, sf):
            pltpu.make_async_copy(
                d_hbm.at[pl.ds(base + j * CHG, CHG)], eb, sf).wait()

        def add(j, eb):
            pltpu.sync_copy(eb, acc.at[idx.at[pl.ds(j * CHG, CHG)]],
                            add=True)

        fetch(0, eba, sfa)

        def body(i, carry):
            j = 2 * i
            fetch(j + 1, ebb, sfb)
            wait_fetch(j, eba, sfa)
            add(j, eba)
            fetch(j + 2, eba, sfa)
            wait_fetch(j + 1, ebb, sfb)
            add(j + 1, ebb)
            return carry

        lax.fori_loop(0, (full - 2) // 2, body, 0)
        fetch(full - 1, ebb, sfb)
        wait_fetch(full - 2, eba, sfa)
        add(full - 2, eba)
        wait_fetch(full - 1, ebb, sfb)
        add(full - 1, ebb)
        if tail:
            to = full * CHG
            pltpu.async_copy(d_hbm.at[pl.ds(base + to, tail)],
                             eba.at[pl.ds(0, tail)], sfa)
            pltpu.make_async_copy(d_hbm.at[pl.ds(base + to, tail)],
                                  eba.at[pl.ds(0, tail)], sfa).wait()
            pltpu.sync_copy(eba.at[pl.ds(0, tail)],
                            acc.at[idx.at[pl.ds(to, tail)]], add=True)
        plsc.subcore_barrier()

        def outj(j, carry):
            r = row0 + j * RC
            pltpu.sync_copy(acc.at[pl.ds(r, RC)], ib)
            pltpu.sync_copy(ib, out.at[cid, pl.ds(r, RC)])
            return carry

        lax.fori_loop(0, n_chunks, outj, 0)

    return scatter_k(data, dst, zeros_w)


def _dot(a, b):
    return jnp.dot(a, b, preferred_element_type=jnp.float32)


def _pack_kv(k, v):
    ku = lax.bitcast_convert_type(k.astype(jnp.bfloat16),
                                  jnp.uint16).astype(jnp.uint32)
    vu = lax.bitcast_convert_type(v.astype(jnp.bfloat16),
                                  jnp.uint16).astype(jnp.uint32)
    return lax.bitcast_convert_type((ku << 16) | vu, jnp.float32)


def _unpack_kv(p):
    u = lax.bitcast_convert_type(p, jnp.uint32)
    kb = lax.bitcast_convert_type((u >> 16).astype(jnp.uint16), jnp.bfloat16)
    vb = lax.bitcast_convert_type(u.astype(jnp.uint16), jnp.bfloat16)
    return kb.astype(jnp.float32), vb.astype(jnp.float32)


def _full(shape=None):
    return pl.BlockSpec(memory_space=pltpu.ANY) if shape is None else \
        pl.BlockSpec(shape, lambda i: (0,) * len(shape))


def _rows(shape):
    return pl.BlockSpec(shape, lambda i: (i,) + (0,) * (len(shape) - 1))


def _tc_embed_qkv(h, embW, embb, qW, qb, kW, kb, vW, vb):
    N = h.shape[0]

    def body(h_ref, ew_ref, eb_ref, qw_ref, qb_ref, kw_ref, kb_ref,
             vw_ref, vb_ref, hh_ref, q_ref, kv_ref):
        hh = _dot(h_ref[...], ew_ref[...]) + eb_ref[...]
        hh_ref[...] = hh
        q_ref[...] = _dot(hh, qw_ref[...]) + qb_ref[...]
        kv_ref[...] = _pack_kv(_dot(hh, kw_ref[...]) + kb_ref[...],
                               _dot(hh, vw_ref[...]) + vb_ref[...])

    return pl.pallas_call(
        body,
        out_shape=[jax.ShapeDtypeStruct((N, D), jnp.float32)] * 3,
    )(h, embW, embb, qW, qb, kW, kb, vW, vb)


def _tc_edge_full(kvsrc, qdst, e, embW, embb, eW, eb, oeW, oeb,
                  ones_hb, ones_hb_t):
    NE = kvsrc.shape[0]
    F = e.shape[1]
    grid = NE // RB

    def body(kv_ref, qd_ref, e_ref, ew_ref, ebias_ref, pw_ref, pb_ref,
             ow_ref, ob_ref, hb_ref, hbt_ref,
             wva_ref, wvb_ref, epre_ref, st_ref):
        ksrc, vsrc = _unpack_kv(kv_ref[...])
        ee = _dot(e_ref[...], ew_ref[...]) + ebias_ref[...]
        ep = _dot(ee, pw_ref[...]) + pb_ref[...]
        score = ksrc * qd_ref[...] * ep * 0.25
        ssum = _dot(score, hbt_ref[...])
        attn = jnp.exp(jnp.clip(ssum, -5.0, 5.0))
        attnb = _dot(attn, hb_ref[...])
        wv = vsrc * attnb
        pad = jnp.zeros((RB, WA - HD - H), jnp.float32)
        wva_ref[...] = jnp.concatenate([wv[:, :HD], attn, pad], axis=1)
        wvb_ref[...] = jnp.concatenate([wv[:, HD:], attn, pad], axis=1)
        epre = ee + _dot(score, ow_ref[...]) + ob_ref[...]
        epre_ref[...] = epre

        @pl.when(pl.program_id(0) == 0)
        def _():
            st_ref[...] = jnp.zeros_like(st_ref)

        s = jnp.sum(epre, axis=0)
        ss = jnp.sum(epre * epre, axis=0)
        st_ref[...] += jnp.concatenate(
            [s[None], ss[None], jnp.zeros((6, D), jnp.float32)], axis=0)

    return pl.pallas_call(
        body,
        grid=(grid,),
        in_specs=[_rows((RB, D)), _rows((RB, D)), _rows((RB, F)),
                  _full((F, D)), _full((1, D)), _full((D, D)), _full((1, D)),
                  _full((D, D)), _full((1, D)), _full((H, D)), _full((D, H))],
        out_specs=[_rows((RB, WA)), _rows((RB, WA)), _rows((RB, D)),
                   _full((8, D))],
        out_shape=[jax.ShapeDtypeStruct((NE, WA), jnp.float32),
                   jax.ShapeDtypeStruct((NE, WA), jnp.float32),
                   jax.ShapeDtypeStruct((NE, D), jnp.float32),
                   jax.ShapeDtypeStruct((8, D), jnp.float32)],
    )(kvsrc, qdst, e, embW, embb, eW, eb, oeW, oeb, ones_hb, ones_hb_t)


def _tc_edge_lite(kvsrc, qdst, y, stats, g2, b2, eW, eb, n_rows,
                  ones_hb, ones_hb_t):
    NE = kvsrc.shape[0]
    grid = NE // RB
    inv_n = 1.0 / float(n_rows)

    def body(kv_ref, qd_ref, y_ref, st_ref, g_ref, b_ref, pw_ref, pb_ref,
             hb_ref, hbt_ref, wva_ref, wvb_ref):
        ksrc, vsrc = _unpack_kv(kv_ref[...])
        mean = st_ref[0:1, :] * inv_n
        var = st_ref[1:2, :] * inv_n - mean * mean
        inv = lax.rsqrt(var + 1e-5)
        x = (y_ref[...] - mean) * inv * g_ref[...] + b_ref[...]
        ep = _dot(x, pw_ref[...]) + pb_ref[...]
        score = ksrc * qd_ref[...] * ep * 0.25
        attn = jnp.exp(jnp.clip(_dot(score, hbt_ref[...]), -5.0, 5.0))
        attnb = _dot(attn, hb_ref[...])
        wv = vsrc * attnb
        pad = jnp.zeros((RB, WA - HD - H), jnp.float32)
        wva_ref[...] = jnp.concatenate([wv[:, :HD], attn, pad], axis=1)
        wvb_ref[...] = jnp.concatenate([wv[:, HD:], attn, pad], axis=1)

    return pl.pallas_call(
        body,
        grid=(grid,),
        in_specs=[_rows((RB, D))] * 3
        + [_full((8, D)), _full((1, D)), _full((1, D)),
           _full((D, D)), _full((1, D)), _full((H, D)), _full((D, H))],
        out_specs=[_rows((RB, WA)), _rows((RB, WA))],
        out_shape=[jax.ShapeDtypeStruct((NE, WA), jnp.float32),
                   jax.ShapeDtypeStruct((NE, WA), jnp.float32)],
    )(kvsrc, qdst, y, stats, g2, b2, eW, eb, ones_hb, ones_hb_t)


def _tc_edge_ffn(epre, stats, g1, b1, w1, bb1, w2, bb2, n_rows):
    NE = epre.shape[0]
    grid = NE // RB
    inv_n = 1.0 / float(n_rows)

    def body(ep_ref, st_ref, g_ref, b_ref, w1_ref, b1_ref, w2_ref, b2_ref,
             y_ref, sy_ref):
        mean = st_ref[0:1, :] * inv_n
        var = st_ref[1:2, :] * inv_n - mean * mean
        inv = lax.rsqrt(var + 1e-5)
        x = (ep_ref[...] - mean) * inv * g_ref[...] + b_ref[...]
        hmid = jnp.maximum(_dot(x, w1_ref[...]) + b1_ref[...], 0.0)
        y = x + _dot(hmid, w2_ref[...]) + b2_ref[...]
        y_ref[...] = y

        @pl.when(pl.program_id(0) == 0)
        def _():
            sy_ref[...] = jnp.zeros_like(sy_ref)

        s = jnp.sum(y, axis=0)
        ss = jnp.sum(y * y, axis=0)
        sy_ref[...] += jnp.concatenate(
            [s[None], ss[None], jnp.zeros((6, D), jnp.float32)], axis=0)

    return pl.pallas_call(
        body,
        grid=(grid,),
        in_specs=[_rows((RB, D)), _full((8, D)), _full((1, D)), _full((1, D)),
                  _full((D, 2 * D)), _full((1, 2 * D)),
                  _full((2 * D, D)), _full((1, D))],
        out_specs=[_rows((RB, D)), _full((8, D))],
        out_shape=[jax.ShapeDtypeStruct((NE, D), jnp.float32),
                   jax.ShapeDtypeStruct((8, D), jnp.float32)],
    )(epre, stats, g1, b1, w1, bb1, w2, bb2)


def _node_update(hpa, hpb, hh_in, lp, ones_hb):
    a = hpa[0] + hpa[1]
    b = hpb[0] + hpb[1]
    wv = jnp.concatenate([a[:, :HD], b[:, :HD]], axis=1)
    z = a[:, HD:HD + H]
    r = 1.0 / (z + 1e-6)
    h_att = wv * _dot(r, ones_hb)
    h_new = _dot(h_att, lp['oh_w']) + lp['oh_b'] + hh_in
    m = jnp.mean(h_new, axis=0, keepdims=True)
    v = jnp.mean(h_new * h_new, axis=0, keepdims=True) - m * m
    h_new = (h_new - m) * lax.rsqrt(v + 1e-5) * lp['g1'] + lp['b1']
    h2 = _dot(jnp.maximum(_dot(h_new, lp['f1w']) + lp['f1b'], 0.0),
              lp['f2w']) + lp['f2b']
    h_new = h_new + h2
    m = jnp.mean(h_new, axis=0, keepdims=True)
    v = jnp.mean(h_new * h_new, axis=0, keepdims=True) - m * m
    return (h_new - m) * lax.rsqrt(v + 1e-5) * lp['g2'] + lp['b2']


def _tc_node_mid(hpart, zpart, hh_in, lw, qW, qb, kW, kb, vW, vb, ones_hb):
    N = hh_in.shape[0]

    def body(hp_ref, zp_ref, hh_ref, ow_ref, ob_ref, g1_ref, b1_ref,
             f1w_ref, f1b_ref, f2w_ref, f2b_ref, g2_ref, b2_ref,
             qw_ref, qb_ref, kw_ref, kb_ref, vw_ref, vb_ref, hb_ref,
             hh_out, q_ref, kv_ref):
        lp = dict(oh_w=ow_ref[...], oh_b=ob_ref[...], g1=g1_ref[...],
                  b1=b1_ref[...], f1w=f1w_ref[...], f1b=f1b_ref[...],
                  f2w=f2w_ref[...], f2b=f2b_ref[...], g2=g2_ref[...],
                  b2=b2_ref[...])
        hh1 = _node_update(hp_ref[...], zp_ref[...], hh_ref[...], lp,
                           hb_ref[...])
        hh_out[...] = hh1
        q_ref[...] = _dot(hh1, qw_ref[...]) + qb_ref[...]
        kv_ref[...] = _pack_kv(_dot(hh1, kw_ref[...]) + kb_ref[...],
                               _dot(hh1, vw_ref[...]) + vb_ref[...])

    return pl.pallas_call(
        body,
        out_shape=[jax.ShapeDtypeStruct((N, D), jnp.float32)] * 3,
    )(hpart, zpart, hh_in, lw['oh_w'], lw['oh_b'], lw['g1'], lw['b1'],
      lw['f1w'], lw['f1b'], lw['f2w'], lw['f2b'], lw['g2'], lw['b2'],
      qW, qb, kW, kb, vW, vb, ones_hb)


def _tc_node_head(hpart, zpart, hh_in, lw, w_top, w_bot, mlp1b, mlp2w,
                  mlp2b, vid, ones_hb, num_nodes):
    N = hh_in.shape[0]
    bs = vid.shape[0]

    def body(hp_ref, zp_ref, hh_ref, ow_ref, ob_ref, g1_ref, b1_ref,
             f1w_ref, f1b_ref, f2w_ref, f2b_ref, g2_ref, b2_ref,
             wt_ref, wb_ref, m1b_ref, m2w_ref, m2b_ref, hb_ref, vid_ref,
             out_ref, hh2_ref):
        lp = dict(oh_w=ow_ref[...], oh_b=ob_ref[...], g1=g1_ref[...],
                  b1=b1_ref[...], f1w=f1w_ref[...], f1b=f1b_ref[...],
                  f2w=f2w_ref[...], f2b=f2b_ref[...], g2=g2_ref[...],
                  b2=b2_ref[...])
        hh2 = _node_update(hp_ref[...], zp_ref[...], hh_ref[...], lp,
                           hb_ref[...])
        hh2_ref[...] = hh2
        for b in range(bs):
            row = hh2_ref[pl.ds(vid_ref[b], 1), :]
            cb = _dot(row, wt_ref[...]) + m1b_ref[...]
            blk = hh2_ref[pl.ds(b * num_nodes, num_nodes), :]
            t = jnp.maximum(_dot(blk, wb_ref[...]) + cb, 0.0)
            out_ref[pl.ds(b * num_nodes, num_nodes), :] = (
                _dot(t, m2w_ref[...]) + m2b_ref[...])

    return pl.pallas_call(
        body,
        in_specs=[pl.BlockSpec(memory_space=pltpu.VMEM)] * 19
        + [pl.BlockSpec(memory_space=pltpu.SMEM)],
        out_specs=pl.BlockSpec(memory_space=pltpu.VMEM),
        out_shape=jax.ShapeDtypeStruct((N, 1), jnp.float32),
        scratch_shapes=[pltpu.VMEM((N, D), jnp.float32)],
    )(hpart, zpart, hh_in, lw['oh_w'], lw['oh_b'], lw['g1'], lw['b1'],
      lw['f1w'], lw['f1b'], lw['f2w'], lw['f2b'], lw['g2'], lw['b2'],
      w_top, w_bot, mlp1b, mlp2w, mlp2b, ones_hb, vid)


def _row(x):
    return x.reshape(1, -1)


def _layer_w(lp):
    return dict(oh_w=lp['Oh_W'], oh_b=_row(lp['Oh_b']),
                g1=_row(lp['bn1h_g']), b1=_row(lp['bn1h_b']),
                f1w=lp['ffh1_W'], f1b=_row(lp['ffh1_b']),
                f2w=lp['ffh2_W'], f2b=_row(lp['ffh2_b']),
                g2=_row(lp['bn2h_g']), b2=_row(lp['bn2h_b']))


def kernel(h, e, params, edge_index, vehicle_node_id, batch_size):
    src = edge_index[0]
    dst = edge_index[1]
    N = h.shape[0]
    NE = src.shape[0]
    bs = vehicle_node_id.shape[0]
    num_nodes = N // bs
    L1, L2 = params['layers']

    ones_hb = jnp.repeat(jnp.eye(H, dtype=jnp.float32), DH, axis=1)
    ones_hb_t = ones_hb.T

    hh0, q1, kv1 = _tc_embed_qkv(
        h, params['emb_h_W'], _row(params['emb_h_b']),
        L1['Q_W'], _row(L1['Q_b']), L1['K_W'], _row(L1['K_b']),
        L1['V_W'], _row(L1['V_b']))

    kvsrc1, qdst1 = _sc_gather(kv1, q1, src, dst)
    wv1a, wv1b, epre1, st1 = _tc_edge_full(
        kvsrc1, qdst1, e, params['emb_e_W'], _row(params['emb_e_b']),
        L1['E_W'], _row(L1['E_b']), L1['Oe_W'], _row(L1['Oe_b']),
        ones_hb, ones_hb_t)
    hpa1 = _sc_scatter(wv1a, dst, N)
    hpb1 = _sc_scatter(wv1b, dst, N)
    hh1, q2, kv2 = _tc_node_mid(
        hpa1, hpb1, hh0, _layer_w(L1),
        L2['Q_W'], _row(L2['Q_b']), L2['K_W'], _row(L2['K_b']),
        L2['V_W'], _row(L2['V_b']), ones_hb)
    y1, sty1 = _tc_edge_ffn(
        epre1, st1, _row(L1['bn1e_g']), _row(L1['bn1e_b']),
        L1['ffe1_W'], _row(L1['ffe1_b']), L1['ffe2_W'], _row(L1['ffe2_b']),
        NE)

    kvsrc2, qdst2 = _sc_gather(kv2, q2, src, dst)
    wv2a, wv2b = _tc_edge_lite(
        kvsrc2, qdst2, y1, sty1, _row(L1['bn2e_g']), _row(L1['bn2e_b']),
        L2['E_W'], _row(L2['E_b']), NE, ones_hb, ones_hb_t)
    hpa2 = _sc_scatter(wv2a, dst, N)
    hpb2 = _sc_scatter(wv2b, dst, N)

    vid = (vehicle_node_id.astype(jnp.int32)
           + jnp.arange(bs, dtype=jnp.int32) * num_nodes
           + (jnp.asarray(batch_size, jnp.int32) - jnp.int32(bs)))
    policy = _tc_node_head(
        hpa2, hpb2, hh1, _layer_w(L2),
        params['mlp1_W'][:D], params['mlp1_W'][D:], _row(params['mlp1_b']),
        params['mlp2_W'], _row(params['mlp2_b']), vid, ones_hb, num_nodes)
    return policy[:, 0].reshape(bs, num_nodes)
